# baseline (speedup 1.0000x reference)
import functools

import jax
import jax.numpy as jnp
from jax import lax
from jax.experimental import pallas as pl
from jax.experimental.pallas import tpu as pltpu
from jax.experimental.pallas import tpu_sc as plsc

_N = 10000
_E = 160000
_H = 128
_L = 3
_NA = 119
_NB = 22
_FG = 39

_NP = 10240
_EP = 163840
_NC = 2
_NS = 16
_NW = _NC * _NS
_CH = 128
_PW = _EP // _NW
_NCHUNK = _PW // _CH
_RPT = _NP // _NS
_SLAB = _RPT // _CH
_BC = 4

_RN = 512
_RE = 512
_GN = _NP // _RN
_GE = _EP // _RE

_f32 = jnp.float32
_i32 = jnp.int32


def _mesh():
    return plsc.VectorSubcoreMesh(core_axis_name="c", subcore_axis_name="s")


def _sc_gather(table, idx):
    D = table.shape[1]
    dt = table.dtype

    @functools.partial(
        pl.kernel,
        mesh=_mesh(),
        out_type=jax.ShapeDtypeStruct((_EP, D), dt),
        scratch_types=[
            pltpu.VMEM((_CH,), _i32),
            pltpu.VMEM((_CH,), _i32),
            pltpu.VMEM((_CH, D), dt),
            pltpu.VMEM((_CH, D), dt),
            pltpu.SemaphoreType.DMA,
            pltpu.SemaphoreType.DMA,
        ],
    )
    def k(table_hbm, idx_hbm, out_hbm, i0, i1, r0, r1, sem_g, sem_w):
        wid = lax.axis_index("s") * _NC + lax.axis_index("c")
        base = pl.multiple_of(wid * _PW, 8)
        idxs = [i0, i1]
        rows = [r0, r1]
        nb = 2

        def g_start(j, b):
            off = pl.multiple_of(base + j * _CH, 8)
            pltpu.sync_copy(idx_hbm.at[pl.ds(off, _CH)], idxs[b])
            pltpu.async_copy(table_hbm.at[idxs[b]], rows[b], sem_g)

        def g_wait(b):
            pltpu.make_async_copy(table_hbm.at[idxs[b]], rows[b], sem_g).wait()

        def w_start(j, b):
            off = pl.multiple_of(base + j * _CH, 8)
            pltpu.async_copy(rows[b], out_hbm.at[pl.ds(off, _CH)], sem_w)

        def w_wait(j, b):
            off = pl.multiple_of(base + j * _CH, 8)
            pltpu.make_async_copy(rows[b], out_hbm.at[pl.ds(off, _CH)], sem_w).wait()

        for b in range(nb):
            g_start(b, b)

        def body(t, carry):
            j0 = t * nb
            for b in range(nb):
                g_wait(b)
                w_start(j0 + b, b)
            for b in range(nb):
                w_wait(j0 + b, b)
                g_start(j0 + nb + b, b)
            return carry

        lax.fori_loop(0, _NCHUNK // nb - 1, body, 0)
        jl = _NCHUNK - nb
        for b in range(nb):
            g_wait(b)
            w_start(jl + b, b)
        for b in range(nb):
            w_wait(jl + b, b)

    return k(table, idx)


def _sc_scatter_add(vals, idx, zeros_h):
    D = vals.shape[1]

    @functools.partial(
        pl.kernel,
        mesh=_mesh(),
        out_type=jax.ShapeDtypeStruct((2 * _NP, D), _f32),
        scratch_types=[
            pltpu.VMEM((_CH,), _i32),
            pltpu.VMEM((_CH,), _i32),
            pltpu.VMEM((_CH, D), _f32),
            pltpu.VMEM((_CH, D), _f32),
            pltpu.VMEM_SHARED((_NP, D), _f32),
            pltpu.SemaphoreType.DMA,
            pltpu.SemaphoreType.DMA,
            pltpu.SemaphoreType.DMA,
        ],
    )
    def k(vals_hbm, idx_hbm, zeros_hbm, out_hbm,
          i0, i1, v0, v1, acc_sh, sem_i, sem_l, sem_s):
        cid = lax.axis_index("c")
        sid = lax.axis_index("s")
        wid = sid * _NC + cid
        r0 = sid * _RPT
        idxs = [i0, i1]
        bufs = [v0, v1]
        nb = 2
        pltpu.sync_copy(zeros_hbm, v0)

        def zbody(kk, carry):
            pltpu.sync_copy(v0, acc_sh.at[pl.ds(pl.multiple_of(r0 + kk * _CH, 8), _CH)])
            return carry

        lax.fori_loop(0, _SLAB, zbody, 0)
        plsc.subcore_barrier()

        base = pl.multiple_of(wid * _PW, 8)

        def l_start(j, b):
            off = pl.multiple_of(base + j * _CH, 8)
            pltpu.async_copy(idx_hbm.at[pl.ds(off, _CH)], idxs[b], sem_i)
            pltpu.async_copy(vals_hbm.at[pl.ds(off, _CH)], bufs[b], sem_l)

        def l_wait(j, b):
            off = pl.multiple_of(base + j * _CH, 8)
            pltpu.make_async_copy(idx_hbm.at[pl.ds(off, _CH)], idxs[b], sem_i).wait()
            pltpu.make_async_copy(vals_hbm.at[pl.ds(off, _CH)], bufs[b], sem_l).wait()

        def s_start(b):
            pltpu.async_copy(bufs[b], acc_sh.at[idxs[b]], sem_s, add=True)

        def s_wait(b):
            pltpu.make_async_copy(bufs[b], acc_sh.at[idxs[b]], sem_s).wait()

        for b in range(nb):
            l_start(b, b)

        def body(t, carry):
            j0 = t * nb
            for b in range(nb):
                l_wait(j0 + b, b)
                s_start(b)
            for b in range(nb):
                s_wait(b)
                l_start(j0 + nb + b, b)
            return carry

        lax.fori_loop(0, _NCHUNK // nb - 1, body, 0)
        jl = _NCHUNK - nb
        for b in range(nb):
            l_wait(jl + b, b)
            s_start(b)
        for b in range(nb):
            s_wait(b)
        plsc.subcore_barrier()

        def obody(kk, carry):
            src_off = pl.multiple_of(r0 + kk * _CH, 8)
            dst_off = pl.multiple_of(cid * _NP + r0 + kk * _CH, 8)
            pltpu.sync_copy(acc_sh.at[pl.ds(src_off, _CH)], v0)
            pltpu.sync_copy(v0, out_hbm.at[pl.ds(dst_off, _CH)])
            return carry

        lax.fori_loop(0, _SLAB, obody, 0)

    return k(vals, idx, zeros_h)


def _sc_gather_scatter(z, u, src, dst, zeros_h):
    D = z.shape[1]
    ch = 64
    nchunk = _PW // ch

    @functools.partial(
        pl.kernel,
        mesh=_mesh(),
        out_type=jax.ShapeDtypeStruct((2 * _NP, D), _f32),
        scratch_types=[
            pltpu.VMEM((ch,), _i32),
            pltpu.VMEM((ch,), _i32),
            pltpu.VMEM((ch,), _i32),
            pltpu.VMEM((ch,), _i32),
            pltpu.VMEM((ch, D), _f32),
            pltpu.VMEM((ch, D), _f32),
            pltpu.VMEM((ch, D), _f32),
            pltpu.VMEM((ch, D), _f32),
            pltpu.VMEM_SHARED((_NP, D), _f32),
            pltpu.SemaphoreType.DMA,
            pltpu.SemaphoreType.DMA,
            pltpu.SemaphoreType.DMA,
        ],
    )
    def k(z_hbm, u_hbm, src_hbm, dst_hbm, zeros_hbm, out_hbm,
          s0, s1, d0, d1, g0, g1, u0, u1, acc_sh, sem_g, sem_l, sem_s):
        cid = lax.axis_index("c")
        sid = lax.axis_index("s")
        wid = sid * _NC + cid
        r0 = sid * _RPT
        sidx = [s0, s1]
        didx = [d0, d1]
        gbufs = [g0, g1]
        ubufs = [u0, u1]
        nb = 2
        pltpu.sync_copy(zeros_hbm.at[pl.ds(0, ch)], g0)

        def zbody(kk, carry):
            pltpu.sync_copy(g0, acc_sh.at[pl.ds(pl.multiple_of(r0 + kk * ch, 8), ch)])
            return carry

        lax.fori_loop(0, _RPT // ch, zbody, 0)
        plsc.subcore_barrier()

        base = pl.multiple_of(wid * _PW, 8)

        def l_start(j, b):
            off = pl.multiple_of(base + j * ch, 8)
            pltpu.sync_copy(src_hbm.at[pl.ds(off, ch)], sidx[b])
            pltpu.sync_copy(dst_hbm.at[pl.ds(off, ch)], didx[b])
            pltpu.async_copy(z_hbm.at[sidx[b]], gbufs[b], sem_g)
            pltpu.async_copy(u_hbm.at[pl.ds(off, ch)], ubufs[b], sem_l)

        def l_wait(j, b):
            off = pl.multiple_of(base + j * ch, 8)
            pltpu.make_async_copy(z_hbm.at[sidx[b]], gbufs[b], sem_g).wait()
            pltpu.make_async_copy(u_hbm.at[pl.ds(off, ch)], ubufs[b], sem_l).wait()

        def s_start(b):
            pltpu.async_copy(gbufs[b], acc_sh.at[didx[b]], sem_s, add=True)
            pltpu.async_copy(ubufs[b], acc_sh.at[didx[b]], sem_s, add=True)

        def s_wait(b):
            pltpu.make_async_copy(gbufs[b], acc_sh.at[didx[b]], sem_s).wait()
            pltpu.make_async_copy(ubufs[b], acc_sh.at[didx[b]], sem_s).wait()

        for b in range(nb):
            l_start(b, b)

        def body(t, carry):
            j0 = t * nb
            for b in range(nb):
                l_wait(j0 + b, b)
                s_start(b)
            for b in range(nb):
                s_wait(b)
                l_start(j0 + nb + b, b)
            return carry

        lax.fori_loop(0, nchunk // nb - 1, body, 0)
        jl = nchunk - nb
        for b in range(nb):
            l_wait(jl + b, b)
            s_start(b)
        for b in range(nb):
            s_wait(b)
        plsc.subcore_barrier()

        def obody(kk, carry):
            src_off = pl.multiple_of(r0 + kk * ch, 8)
            dst_off = pl.multiple_of(cid * _NP + r0 + kk * ch, 8)
            pltpu.sync_copy(acc_sh.at[pl.ds(src_off, ch)], g0)
            pltpu.sync_copy(g0, out_hbm.at[pl.ds(dst_off, ch)])
            return carry

        lax.fori_loop(0, _RPT // ch, obody, 0)

    return k(z, u, src, dst, zeros_h)


def _cparams():
    return pltpu.CompilerParams(dimension_semantics=("arbitrary",))


def _onehot_dot(idx, table_ref, ncls):
    oh = (idx[:, None] == lax.broadcasted_iota(_i32, (idx.shape[0], ncls), 1)).astype(_f32)
    return jax.lax.dot_general(oh, table_ref[...], (((1,), (0,)), ((), ())),
                               precision=jax.lax.Precision.HIGHEST,
                               preferred_element_type=_f32)


def _embed_nodes(x0, x1, nmw, nmm, a0p, a1p, wm0):
    def body(x0_r, x1_r, mw_r, mm_r, a0_r, a1_r, wm_r, ow_r, om_r, ozw_r, ozm_r):
        i0 = x0_r[0, 0, :]
        i1 = x1_r[0, 0, :]
        h = _onehot_dot(i0, a0_r, 128) + _onehot_dot(i1, a1_r, 128)
        hw = h * (1.0 - mw_r[0, 0, :])[:, None]
        hm = h * (1.0 - mm_r[0, 0, :])[:, None]
        ow_r[...] = hw
        om_r[...] = hm
        ozw_r[...] = jnp.dot(hw, wm_r[...], preferred_element_type=_f32)
        ozm_r[...] = jnp.dot(hm, wm_r[...], preferred_element_type=_f32)

    sh = jax.ShapeDtypeStruct((_NP, _H), _f32)
    v3 = pl.BlockSpec((1, 1, _RN), lambda i: (i, 0, 0))
    full = pl.BlockSpec((128, _H), lambda i: (0, 0))
    blkh = pl.BlockSpec((_RN, _H), lambda i: (i, 0))
    return pl.pallas_call(
        body, grid=(_GN,),
        in_specs=[v3, v3, v3, v3, full, full, full],
        out_specs=[blkh] * 4,
        out_shape=[sh] * 4, compiler_params=_cparams(),
    )(x0, x1, nmw, nmm, a0p, a1p, wm0)


def _embed_edges(e0, e1, emw, emm, b0p, b1p, wm0):
    def body(e0_r, e1_r, mw_r, mm_r, b0_r, b1_r, wm_r, ow_r, om_r, ouw_r, oum_r):
        i0 = e0_r[0, 0, :]
        i1 = e1_r[0, 0, :]
        h = _onehot_dot(i0, b0_r, 32) + _onehot_dot(i1, b1_r, 32)
        ew = h * (1.0 - mw_r[0, 0, :])[:, None]
        em = h * (1.0 - mm_r[0, 0, :])[:, None]
        ow_r[...] = ew
        om_r[...] = em
        ouw_r[...] = jnp.dot(ew, wm_r[...], preferred_element_type=_f32)
        oum_r[...] = jnp.dot(em, wm_r[...], preferred_element_type=_f32)

    sh = jax.ShapeDtypeStruct((_EP, _H), _f32)
    v3 = pl.BlockSpec((1, 1, _RE), lambda i: (i, 0, 0))
    full = pl.BlockSpec((32, _H), lambda i: (0, 0))
    wfull = pl.BlockSpec((_H, _H), lambda i: (0, 0))
    blk = pl.BlockSpec((_RE, _H), lambda i: (i, 0))
    return pl.pallas_call(
        body, grid=(_GE,),
        in_specs=[v3, v3, v3, v3, full, full, wfull],
        out_specs=[blk] * 4,
        out_shape=[sh] * 4, compiler_params=_cparams(),
    )(e0, e1, emw, emm, b0p, b1p, wm0)


def _node_update(h, parts, degp, w, wm_next, emit_z):
    def body(h_r, p0_r, p1_r, d0_r, d1_r, w_r, wm_r, o_r, *outs):
        agg = p0_r[...] + p1_r[...]
        deg = d0_r[:, 0:1] + d1_r[:, 0:1]
        inv = 1.0 / jnp.maximum(deg, 1.0)
        hn = jnp.maximum(
            jnp.dot(h_r[...], w_r[...], preferred_element_type=_f32) + agg * inv, 0.0)
        o_r[...] = hn
        if emit_z:
            outs[0][...] = jnp.dot(hn, wm_r[...], preferred_element_type=_f32)

    blk = pl.BlockSpec((_RN, _H), lambda i: (i, 0))
    p0 = pl.BlockSpec((_RN, _H), lambda i: (i, 0))
    p1 = pl.BlockSpec((_RN, _H), lambda i: (i + _GN, 0))
    d0 = pl.BlockSpec((_RN, _H), lambda i: (i, 0))
    d1 = pl.BlockSpec((_RN, _H), lambda i: (i + _GN, 0))
    full = pl.BlockSpec((_H, _H), lambda i: (0, 0))
    sh = jax.ShapeDtypeStruct((_NP, _H), _f32)
    out_specs = [blk] + ([blk] if emit_z else [])
    out_shape = [sh] + ([sh] if emit_z else [])
    return pl.pallas_call(
        body, grid=(_GN,), in_specs=[blk, p0, p1, d0, d1, full, full],
        out_specs=out_specs, out_shape=out_shape,
        compiler_params=_cparams(),
    )(h, parts, parts, degp, degp, w, wm_next)


def _edge_update(ea, ga, gb, w, wm_next, emit_u):
    def body(ea_r, ga_r, gb_r, w_r, wm_r, o_r, *outs):
        en = jnp.maximum(
            jnp.dot(ea_r[...], w_r[...], preferred_element_type=_f32)
            + ga_r[...] + gb_r[...], 0.0)
        o_r[...] = en
        if emit_u:
            outs[0][...] = jnp.dot(en, wm_r[...], preferred_element_type=_f32)

    blk = pl.BlockSpec((_RE, _H), lambda i: (i, 0))
    full = pl.BlockSpec((_H, _H), lambda i: (0, 0))
    sh = jax.ShapeDtypeStruct((_EP, _H), _f32)
    out_specs = [blk] + ([blk] if emit_u else [])
    out_shape = [sh] + ([sh] if emit_u else [])
    return pl.pallas_call(
        body, grid=(_GE,), in_specs=[blk, blk, blk, full, full],
        out_specs=out_specs, out_shape=out_shape, compiler_params=_cparams(),
    )(ea, ga, gb, w, wm_next)


def _acc2(ref, val, i):
    @pl.when(i == 0)
    def _():
        ref[...] = jnp.zeros((1, 1), _f32)

    ref[...] += jnp.reshape(val, (1, 1))


def _ce_from_logits(z, lab, ncls_real):
    cm = lax.broadcasted_iota(_i32, z.shape, 1) < ncls_real
    z = jnp.where(cm, z, -1e30)
    m = jnp.max(z, axis=1, keepdims=True)
    lse = m + jnp.log(jnp.sum(jnp.exp(z - m), axis=1, keepdims=True))
    oh = lab[:, None] == lax.broadcasted_iota(_i32, z.shape, 1)
    zy = jnp.sum(jnp.where(oh, z, 0.0), axis=1, keepdims=True)
    return lse - zy


def _node_ce(x1, x2, x3, wn3, bn, lab3, w3):
    def body(x1_r, x2_r, x3_r, w_r, b_r, lab_r, wt_r, s_r, sw_r):
        i = pl.program_id(0)
        z = (jnp.dot(x1_r[...], w_r[0], preferred_element_type=_f32)
             + jnp.dot(x2_r[...], w_r[1], preferred_element_type=_f32)
             + jnp.dot(x3_r[...], w_r[2], preferred_element_type=_f32)
             + b_r[...])
        nll = _ce_from_logits(z, lab_r[0, 0, :], _NA)
        wt = wt_r[0, 0, :][:, None]
        _acc2(s_r, jnp.sum(nll * wt), i)
        _acc2(sw_r, jnp.sum(wt), i)

    blk = pl.BlockSpec((_RN, _H), lambda i: (i, 0))
    wfull = pl.BlockSpec((3, _H, 128), lambda i: (0, 0, 0))
    bfull = pl.BlockSpec((1, 128), lambda i: (0, 0))
    v3 = pl.BlockSpec((1, 1, _RN), lambda i: (i, 0, 0))
    sc = pl.BlockSpec((1, 1), lambda i: (0, 0))
    s1 = jax.ShapeDtypeStruct((1, 1), _f32)
    return pl.pallas_call(
        body, grid=(_GN,), in_specs=[blk, blk, blk, wfull, bfull, v3, v3],
        out_specs=[sc, sc], out_shape=[s1, s1], compiler_params=_cparams(),
    )(x1, x2, x3, wn3, bn, lab3, w3)


def _edge_ce(es, gas, gbs, we3, be, lab3, w3, emit_e):
    def body(e1_r, e2_r, e3_r, a1_r, a2_r, a3_r, b1_r, b2_r, b3_r,
             w_r, b_r, lab_r, wt_r, *outs):
        i = pl.program_id(0)
        c1 = e1_r[...] + a1_r[...] + b1_r[...]
        c2 = e2_r[...] + a2_r[...] + b2_r[...]
        c3 = e3_r[...] + a3_r[...] + b3_r[...]
        z = (jnp.dot(c1, w_r[0], preferred_element_type=_f32)
             + jnp.dot(c2, w_r[1], preferred_element_type=_f32)
             + jnp.dot(c3, w_r[2], preferred_element_type=_f32)
             + b_r[...])
        nll = _ce_from_logits(z, lab_r[0, 0, :], _NB)
        wt = wt_r[0, 0, :][:, None]
        _acc2(outs[0], jnp.sum(nll * wt), i)
        _acc2(outs[1], jnp.sum(wt), i)
        if emit_e:
            outs[2][...] = c1
            outs[3][...] = c2
            outs[4][...] = c3

    blk = pl.BlockSpec((_RE, _H), lambda i: (i, 0))
    wfull = pl.BlockSpec((3, _H, 32), lambda i: (0, 0, 0))
    bfull = pl.BlockSpec((1, 32), lambda i: (0, 0))
    v3 = pl.BlockSpec((1, 1, _RE), lambda i: (i, 0, 0))
    sc = pl.BlockSpec((1, 1), lambda i: (0, 0))
    s1 = jax.ShapeDtypeStruct((1, 1), _f32)
    eo = jax.ShapeDtypeStruct((_EP, _H), _f32)
    out_specs = [sc, sc] + ([blk] * 3 if emit_e else [])
    out_shape = [s1, s1] + ([eo] * 3 if emit_e else [])
    return pl.pallas_call(
        body, grid=(_GE,),
        in_specs=[blk] * 9 + [wfull, bfull, v3, v3],
        out_specs=out_specs, out_shape=out_shape, compiler_params=_cparams(),
    )(es[0], es[1], es[2], gas[0], gas[1], gas[2], gbs[0], gbs[1], gbs[2],
      we3, be, lab3, w3)


def _bce_elem(z, y):
    return jnp.maximum(z, 0.0) - z * y + jnp.log(1.0 + jnp.exp(-jnp.abs(z)))


def _node_comp(x1, x2, x3, wf0, bf0, wf1, bf1, ws0, bs0, ws1, bs1,
               labfg, labsc3, nmm13):
    def body(x1_r, x2_r, x3_r, wf0_r, bf0_r, wf1_r, bf1_r,
             ws0_r, bs0_r, ws1_r, bs1_r, yfg_r, ysc_r, m_r,
             sfg_r, ssc_r, sw_r):
        i = pl.program_id(0)
        w = (1.0 - m_r[0, 0, :])[:, None]

        hf = (jnp.dot(x1_r[...], wf0_r[0], preferred_element_type=_f32)
              + jnp.dot(x2_r[...], wf0_r[1], preferred_element_type=_f32)
              + jnp.dot(x3_r[...], wf0_r[2], preferred_element_type=_f32)
              + bf0_r[...])
        hf = jnp.maximum(hf, 0.0)
        zf = jnp.dot(hf, wf1_r[...], preferred_element_type=_f32) + bf1_r[...]
        cmf = (lax.broadcasted_iota(_i32, zf.shape, 1) < _FG).astype(_f32)
        perf = jnp.sum(_bce_elem(zf, yfg_r[...]) * cmf, axis=1, keepdims=True) / _FG

        hs = (jnp.dot(x1_r[...], ws0_r[0], preferred_element_type=_f32)
              + jnp.dot(x2_r[...], ws0_r[1], preferred_element_type=_f32)
              + jnp.dot(x3_r[...], ws0_r[2], preferred_element_type=_f32)
              + bs0_r[...])
        hs = jnp.maximum(hs, 0.0)
        zs = jnp.dot(hs, ws1_r[...], preferred_element_type=_f32) + bs1_r[...]
        ysc = ysc_r[0, 0, :][:, None]
        pers = _bce_elem(zs[:, 0:1], ysc)
        _acc2(sfg_r, jnp.sum(perf * w), i)
        _acc2(ssc_r, jnp.sum(pers * w), i)
        _acc2(sw_r, jnp.sum(w), i)

    blk = pl.BlockSpec((_RN, _H), lambda i: (i, 0))
    w0full = pl.BlockSpec((3, _H, 384), lambda i: (0, 0, 0))
    b0full = pl.BlockSpec((1, 384), lambda i: (0, 0))
    w1full = pl.BlockSpec((384, 128), lambda i: (0, 0))
    b1full = pl.BlockSpec((1, 128), lambda i: (0, 0))
    yfull = pl.BlockSpec((_RN, 128), lambda i: (i, 0))
    v3 = pl.BlockSpec((1, 1, _RN), lambda i: (i, 0, 0))
    sc = pl.BlockSpec((1, 1), lambda i: (0, 0))
    s1 = jax.ShapeDtypeStruct((1, 1), _f32)
    return pl.pallas_call(
        body, grid=(_GN,),
        in_specs=[blk, blk, blk, w0full, b0full, w1full, b1full,
                  w0full, b0full, w1full, b1full, yfull, v3, v3],
        out_specs=[sc, sc, sc], out_shape=[s1, s1, s1], compiler_params=_cparams(),
    )(x1, x2, x3, wf0, bf0, wf1, bf1, ws0, bs0, ws1, bs1, labfg, labsc3, nmm13)


def _edge_comp(e1, e2, e3, wb0, bb0, wb1, bb1, labbr3, emm13):
    def body(e1_r, e2_r, e3_r, w0_r, b0_r, w1_r, b1_r, y_r, m_r, s_r, sw_r):
        i = pl.program_id(0)
        w = (1.0 - m_r[0, 0, :])[:, None]
        hh = (jnp.dot(e1_r[...], w0_r[0], preferred_element_type=_f32)
              + jnp.dot(e2_r[...], w0_r[1], preferred_element_type=_f32)
              + jnp.dot(e3_r[...], w0_r[2], preferred_element_type=_f32)
              + b0_r[...])
        hh = jnp.maximum(hh, 0.0)
        z = jnp.dot(hh, w1_r[...], preferred_element_type=_f32) + b1_r[...]
        y = y_r[0, 0, :][:, None]
        per = _bce_elem(z[:, 0:1], y)
        _acc2(s_r, jnp.sum(per * w), i)
        _acc2(sw_r, jnp.sum(w), i)

    blk = pl.BlockSpec((_RE, _H), lambda i: (i, 0))
    w0full = pl.BlockSpec((3, _H, 384), lambda i: (0, 0, 0))
    b0full = pl.BlockSpec((1, 384), lambda i: (0, 0))
    w1full = pl.BlockSpec((384, 128), lambda i: (0, 0))
    b1full = pl.BlockSpec((1, 128), lambda i: (0, 0))
    v3 = pl.BlockSpec((1, 1, _RE), lambda i: (i, 0, 0))
    sc = pl.BlockSpec((1, 1), lambda i: (0, 0))
    s1 = jax.ShapeDtypeStruct((1, 1), _f32)
    return pl.pallas_call(
        body, grid=(_GE,),
        in_specs=[blk, blk, blk, w0full, b0full, w1full, b1full, v3, v3],
        out_specs=[sc, sc], out_shape=[s1, s1], compiler_params=_cparams(),
    )(e1, e2, e3, wb0, bb0, wb1, bb1, labbr3, emm13)


def _padn(v, fill=0):
    return jnp.pad(v, ((0, _NP - _N),) + ((0, 0),) * (v.ndim - 1), constant_values=fill)


def _pade(v, fill=0):
    return jnp.pad(v, ((0, _EP - _E),) + ((0, 0),) * (v.ndim - 1), constant_values=fill)


def _r3(v, r):
    return v.reshape(-1, 1, r)


def kernel(x, edge_attr, edge_index, node_mask, edge_mask, node_mask_motif,
           edge_mask_motif, label_fg, label_brics, label_scaffold, params):
    p = params

    x0 = _r3(_padn(x[:, 0].astype(_i32)), _RN)
    x1 = _r3(_padn(x[:, 1].astype(_i32)), _RN)
    e0 = _r3(_pade(edge_attr[:, 0].astype(_i32)), _RE)
    e1 = _r3(_pade(edge_attr[:, 1].astype(_i32)), _RE)
    src = _pade(edge_index[0].astype(_i32), 0)
    dst = _pade(edge_index[1].astype(_i32), _NP - 1)

    nmw = _r3(_padn(node_mask.astype(_f32)), _RN)
    nmm0 = _r3(_padn(node_mask_motif.astype(_f32)), _RN)
    nmm1 = _r3(_padn(node_mask_motif.astype(_f32), 1.0), _RN)
    emw = _r3(_pade(edge_mask.astype(_f32)), _RE)
    emm0 = _r3(_pade(edge_mask_motif.astype(_f32)), _RE)
    emm1 = _r3(_pade(edge_mask_motif.astype(_f32), 1.0), _RE)

    labn = x0
    labe = e0
    labfg = jnp.pad(_padn(label_fg), ((0, 0), (0, 128 - _FG)))
    labsc = _r3(_padn(label_scaffold[:, 0]), _RN)
    labbr = _r3(_pade(label_brics[:, 0]), _RE)

    a0p = jnp.pad(p['emb_a0'], ((0, 128 - _NA), (0, 0)))
    a1p = jnp.pad(p['emb_a1'], ((0, 128 - _NA), (0, 0)))
    b0p = jnp.pad(p['emb_b0'], ((0, 32 - _NB), (0, 0)))
    b1p = jnp.pad(p['emb_b1'], ((0, 32 - _NB), (0, 0)))

    wn3 = jnp.pad(p['Wn'], ((0, 0), (0, 128 - _NA))).reshape(3, _H, 128)
    bn = jnp.pad(p['bn'], (0, 128 - _NA)).reshape(1, 128)
    we3 = jnp.pad(p['We'], ((0, 0), (0, 32 - _NB))).reshape(3, _H, 32)
    be = jnp.pad(p['be'], (0, 32 - _NB)).reshape(1, 32)

    wf0 = p['W_fg_0'].reshape(3, _H, 384)
    bf0 = p['b_fg_0'].reshape(1, 384)
    wf1 = jnp.pad(p['W_fg_1'], ((0, 0), (0, 128 - _FG)))
    bf1 = jnp.pad(p['b_fg_1'], (0, 128 - _FG)).reshape(1, 128)
    ws0 = p['W_scaffold_0'].reshape(3, _H, 384)
    bs0 = p['b_scaffold_0'].reshape(1, 384)
    ws1 = jnp.pad(p['W_scaffold_1'], ((0, 0), (0, 127)))
    bs1 = jnp.pad(p['b_scaffold_1'], (0, 127)).reshape(1, 128)
    wb0 = p['W_brics_0'].reshape(3, _H, 384)
    bb0 = p['b_brics_0'].reshape(1, 384)
    wb1 = jnp.pad(p['W_brics_1'], ((0, 0), (0, 127)))
    bb1 = jnp.pad(p['b_brics_1'], (0, 127)).reshape(1, 128)

    zeros_h = jnp.zeros((_CH, _H), _f32)

    wm = [p['W_msg%d' % i] for i in range(_L)]
    hw, hm, zw, zm = _embed_nodes(x0, x1, nmw, nmm0, a0p, a1p, wm[0])
    eaw, eam, uw, um = _embed_edges(e0, e1, emw, emm0, b0p, b1p, wm[0])
    degp = _sc_scatter_add(jnp.ones((_EP, _H), _f32), dst, zeros_h)

    def run_pass(h, z, ea, u, emit_e, nmask, emask):
        xs, gas, gbs, es = [], [], [], []
        for i in range(_L):
            emit = i < _L - 1
            wm_next = wm[i + 1] if emit else wm[i]
            parts = _sc_gather_scatter(z, u, src, dst, zeros_h)
            nouts = _node_update(h, parts, degp, p['W_self%d' % i], wm_next, emit)
            h = nouts[0]
            ga = _sc_gather(h, src)
            gb = _sc_gather(h, dst)
            uouts = _edge_update(ea, ga, gb, p['W_edge%d' % i], wm_next, emit)
            ea = uouts[0]
            if emit:
                z, u = nouts[1], uouts[1]
            xs.append(h); es.append(ea); gas.append(ga); gbs.append(gb)
        sn, swn = _node_ce(xs[0], xs[1], xs[2], wn3, bn, labn, nmask)
        eouts = _edge_ce(es, gas, gbs, we3, be, labe, emask, emit_e)
        return xs, eouts, sn, swn

    xs_w, eo_w, sn_w, swn_w = run_pass(hw, zw, eaw, uw, False, nmw, emw)
    se_w, swe_w = eo_w[0], eo_w[1]
    xs_m, eo_m, sn_m, swn_m = run_pass(hm, zm, eam, um, True, nmm0, emm0)
    se_m, swe_m = eo_m[0], eo_m[1]
    ec1, ec2, ec3 = eo_m[2], eo_m[3], eo_m[4]

    sfg, ssc, swc = _node_comp(xs_m[0], xs_m[1], xs_m[2], wf0, bf0, wf1, bf1,
                               ws0, bs0, ws1, bs1, labfg, labsc, nmm1)
    sbr, swbr = _edge_comp(ec1, ec2, ec3, wb0, bb0, wb1, bb1, labbr, emm1)

    def _div(a, b):
        return (a[0, 0] / jnp.maximum(b[0, 0], 1.0)).astype(_f32)

    ln_w = _div(sn_w, swn_w)
    le_w = _div(se_w, swe_w)
    ln_m = _div(sn_m, swn_m)
    le_m = _div(se_m, swe_m)
    l_fg = _div(sfg, swc)
    l_sc = _div(ssc, swc)
    l_br = _div(sbr, swbr)

    sep = jnp.stack([ln_w, le_w, ln_m, le_m, l_fg, l_br, l_sc])
    loss = ln_w + le_w + ln_m + le_m + l_fg + l_br + l_sc
    return (loss, sep)


# speedup vs baseline: 1.1257x; 1.1257x over previous
import functools

import jax
import jax.numpy as jnp
from jax import lax
from jax.experimental import pallas as pl
from jax.experimental.pallas import tpu as pltpu
from jax.experimental.pallas import tpu_sc as plsc

_N = 10000
_E = 160000
_H = 128
_L = 3
_NA = 119
_NB = 22
_FG = 39

_NP = 10240
_EP = 163840
_NC = 2
_NS = 16
_NW = _NC * _NS
_CH = 128
_PW = _EP // _NW
_NCHUNK = _PW // _CH
_RPT = _NP // _NS
_SLAB = _RPT // _CH
_BC = 4

_RN = 512
_RE = 512
_GN = _NP // _RN
_GE = _EP // _RE

_f32 = jnp.float32
_i32 = jnp.int32


def _mesh():
    return plsc.VectorSubcoreMesh(core_axis_name="c", subcore_axis_name="s")


def _sc_gather(table, idx2):
    D = table.shape[1]
    dt = table.dtype
    R = idx2.shape[0]
    rpw = R // _NW
    nb = 4

    @functools.partial(
        pl.kernel,
        mesh=_mesh(),
        out_type=jax.ShapeDtypeStruct((R * _CH, D), dt),
        scratch_types=[pltpu.VMEM((rpw, _CH), _i32)]
        + [pltpu.VMEM((_CH, D), dt)] * nb
        + [pltpu.SemaphoreType.DMA, pltpu.SemaphoreType.DMA],
    )
    def k(table_hbm, idx_hbm, out_hbm, idx_all, *rest):
        rows = list(rest[:nb])
        sem_g, sem_w = rest[nb], rest[nb + 1]
        wid = lax.axis_index("s") * _NC + lax.axis_index("c")
        irow0 = wid * rpw
        base = pl.multiple_of(wid * rpw * _CH, 8)
        pltpu.sync_copy(idx_hbm.at[pl.ds(irow0, rpw)], idx_all)

        def g_start(j, b):
            pltpu.async_copy(table_hbm.at[idx_all.at[j]], rows[b], sem_g)

        def g_wait(j, b):
            pltpu.make_async_copy(table_hbm.at[idx_all.at[j]], rows[b], sem_g).wait()

        def w_start(j, b):
            off = pl.multiple_of(base + j * _CH, 8)
            pltpu.async_copy(rows[b], out_hbm.at[pl.ds(off, _CH)], sem_w)

        def w_wait(j, b):
            off = pl.multiple_of(base + j * _CH, 8)
            pltpu.make_async_copy(rows[b], out_hbm.at[pl.ds(off, _CH)], sem_w).wait()

        for b in range(nb):
            g_start(b, b)

        def body(t, carry):
            j0 = t * nb
            for b in range(nb):
                g_wait(j0 + b, b)
                w_start(j0 + b, b)
            for b in range(nb):
                w_wait(j0 + b, b)
                g_start(j0 + nb + b, b)
            return carry

        lax.fori_loop(0, rpw // nb - 1, body, 0)
        jl = rpw - nb
        for b in range(nb):
            g_wait(jl + b, b)
            w_start(jl + b, b)
        for b in range(nb):
            w_wait(jl + b, b)

    return k(table, idx2)


def _sc_scatter_add(vals, idx, zeros_h):
    D = vals.shape[1]

    @functools.partial(
        pl.kernel,
        mesh=_mesh(),
        out_type=jax.ShapeDtypeStruct((2 * _NP, D), _f32),
        scratch_types=[
            pltpu.VMEM((_CH,), _i32),
            pltpu.VMEM((_CH,), _i32),
            pltpu.VMEM((_CH, D), _f32),
            pltpu.VMEM((_CH, D), _f32),
            pltpu.VMEM_SHARED((_NP, D), _f32),
            pltpu.SemaphoreType.DMA,
            pltpu.SemaphoreType.DMA,
            pltpu.SemaphoreType.DMA,
        ],
    )
    def k(vals_hbm, idx_hbm, zeros_hbm, out_hbm,
          i0, i1, v0, v1, acc_sh, sem_i, sem_l, sem_s):
        cid = lax.axis_index("c")
        sid = lax.axis_index("s")
        wid = sid * _NC + cid
        r0 = sid * _RPT
        idxs = [i0, i1]
        bufs = [v0, v1]
        nb = 2
        pltpu.sync_copy(zeros_hbm, v0)

        def zbody(kk, carry):
            pltpu.sync_copy(v0, acc_sh.at[pl.ds(pl.multiple_of(r0 + kk * _CH, 8), _CH)])
            return carry

        lax.fori_loop(0, _SLAB, zbody, 0)
        plsc.subcore_barrier()

        base = pl.multiple_of(wid * _PW, 8)

        def l_start(j, b):
            off = pl.multiple_of(base + j * _CH, 8)
            pltpu.async_copy(idx_hbm.at[pl.ds(off, _CH)], idxs[b], sem_i)
            pltpu.async_copy(vals_hbm.at[pl.ds(off, _CH)], bufs[b], sem_l)

        def l_wait(j, b):
            off = pl.multiple_of(base + j * _CH, 8)
            pltpu.make_async_copy(idx_hbm.at[pl.ds(off, _CH)], idxs[b], sem_i).wait()
            pltpu.make_async_copy(vals_hbm.at[pl.ds(off, _CH)], bufs[b], sem_l).wait()

        def s_start(b):
            pltpu.async_copy(bufs[b], acc_sh.at[idxs[b]], sem_s, add=True)

        def s_wait(b):
            pltpu.make_async_copy(bufs[b], acc_sh.at[idxs[b]], sem_s).wait()

        for b in range(nb):
            l_start(b, b)

        def body(t, carry):
            j0 = t * nb
            for b in range(nb):
                l_wait(j0 + b, b)
                s_start(b)
            for b in range(nb):
                s_wait(b)
                l_start(j0 + nb + b, b)
            return carry

        lax.fori_loop(0, _NCHUNK // nb - 1, body, 0)
        jl = _NCHUNK - nb
        for b in range(nb):
            l_wait(jl + b, b)
            s_start(b)
        for b in range(nb):
            s_wait(b)
        plsc.subcore_barrier()

        def obody(kk, carry):
            src_off = pl.multiple_of(r0 + kk * _CH, 8)
            dst_off = pl.multiple_of(cid * _NP + r0 + kk * _CH, 8)
            pltpu.sync_copy(acc_sh.at[pl.ds(src_off, _CH)], v0)
            pltpu.sync_copy(v0, out_hbm.at[pl.ds(dst_off, _CH)])
            return carry

        lax.fori_loop(0, _SLAB, obody, 0)

    return k(vals, idx, zeros_h)


def _sc_gather_scatter(z, u, src, dst, zeros_h):
    D = z.shape[1]
    ch = 64
    nchunk = _PW // ch

    @functools.partial(
        pl.kernel,
        mesh=_mesh(),
        out_type=jax.ShapeDtypeStruct((2 * _NP, D), _f32),
        scratch_types=[
            pltpu.VMEM((ch,), _i32),
            pltpu.VMEM((ch,), _i32),
            pltpu.VMEM((ch,), _i32),
            pltpu.VMEM((ch,), _i32),
            pltpu.VMEM((ch, D), _f32),
            pltpu.VMEM((ch, D), _f32),
            pltpu.VMEM((ch, D), _f32),
            pltpu.VMEM((ch, D), _f32),
            pltpu.VMEM_SHARED((_NP, D), _f32),
            pltpu.SemaphoreType.DMA,
            pltpu.SemaphoreType.DMA,
            pltpu.SemaphoreType.DMA,
        ],
    )
    def k(z_hbm, u_hbm, src_hbm, dst_hbm, zeros_hbm, out_hbm,
          s0, s1, d0, d1, g0, g1, u0, u1, acc_sh, sem_g, sem_l, sem_s):
        cid = lax.axis_index("c")
        sid = lax.axis_index("s")
        wid = sid * _NC + cid
        r0 = sid * _RPT
        sidx = [s0, s1]
        didx = [d0, d1]
        gbufs = [g0, g1]
        ubufs = [u0, u1]
        nb = 2
        pltpu.sync_copy(zeros_hbm.at[pl.ds(0, ch)], g0)

        def zbody(kk, carry):
            pltpu.sync_copy(g0, acc_sh.at[pl.ds(pl.multiple_of(r0 + kk * ch, 8), ch)])
            return carry

        lax.fori_loop(0, _RPT // ch, zbody, 0)
        plsc.subcore_barrier()

        base = pl.multiple_of(wid * _PW, 8)

        def l_start(j, b):
            off = pl.multiple_of(base + j * ch, 8)
            pltpu.sync_copy(src_hbm.at[pl.ds(off, ch)], sidx[b])
            pltpu.sync_copy(dst_hbm.at[pl.ds(off, ch)], didx[b])
            pltpu.async_copy(z_hbm.at[sidx[b]], gbufs[b], sem_g)
            pltpu.async_copy(u_hbm.at[pl.ds(off, ch)], ubufs[b], sem_l)

        def l_wait(j, b):
            off = pl.multiple_of(base + j * ch, 8)
            pltpu.make_async_copy(z_hbm.at[sidx[b]], gbufs[b], sem_g).wait()
            pltpu.make_async_copy(u_hbm.at[pl.ds(off, ch)], ubufs[b], sem_l).wait()

        def s_start(b):
            pltpu.async_copy(gbufs[b], acc_sh.at[didx[b]], sem_s, add=True)
            pltpu.async_copy(ubufs[b], acc_sh.at[didx[b]], sem_s, add=True)

        def s_wait(b):
            pltpu.make_async_copy(gbufs[b], acc_sh.at[didx[b]], sem_s).wait()
            pltpu.make_async_copy(ubufs[b], acc_sh.at[didx[b]], sem_s).wait()

        for b in range(nb):
            l_start(b, b)

        def body(t, carry):
            j0 = t * nb
            for b in range(nb):
                l_wait(j0 + b, b)
                s_start(b)
            for b in range(nb):
                s_wait(b)
                l_start(j0 + nb + b, b)
            return carry

        lax.fori_loop(0, nchunk // nb - 1, body, 0)
        jl = nchunk - nb
        for b in range(nb):
            l_wait(jl + b, b)
            s_start(b)
        for b in range(nb):
            s_wait(b)
        plsc.subcore_barrier()

        def obody(kk, carry):
            src_off = pl.multiple_of(r0 + kk * ch, 8)
            dst_off = pl.multiple_of(cid * _NP + r0 + kk * ch, 8)
            pltpu.sync_copy(acc_sh.at[pl.ds(src_off, ch)], g0)
            pltpu.sync_copy(g0, out_hbm.at[pl.ds(dst_off, ch)])
            return carry

        lax.fori_loop(0, _RPT // ch, obody, 0)

    return k(z, u, src, dst, zeros_h)


def _cparams():
    return pltpu.CompilerParams(dimension_semantics=("arbitrary",))


def _onehot_dot(idx, table_ref, ncls):
    oh = (idx[:, None] == lax.broadcasted_iota(_i32, (idx.shape[0], ncls), 1)).astype(_f32)
    return jax.lax.dot_general(oh, table_ref[...], (((1,), (0,)), ((), ())),
                               precision=jax.lax.Precision.HIGHEST,
                               preferred_element_type=_f32)


def _embed_nodes(x0, x1, nmw, nmm, a0p, a1p, wm0):
    def body(x0_r, x1_r, mw_r, mm_r, a0_r, a1_r, wm_r, ow_r, om_r, ozw_r, ozm_r):
        i0 = x0_r[0, 0, :]
        i1 = x1_r[0, 0, :]
        h = _onehot_dot(i0, a0_r, 128) + _onehot_dot(i1, a1_r, 128)
        hw = h * (1.0 - mw_r[0, 0, :])[:, None]
        hm = h * (1.0 - mm_r[0, 0, :])[:, None]
        ow_r[...] = hw
        om_r[...] = hm
        ozw_r[...] = jnp.dot(hw, wm_r[...], preferred_element_type=_f32)
        ozm_r[...] = jnp.dot(hm, wm_r[...], preferred_element_type=_f32)

    sh = jax.ShapeDtypeStruct((_NP, _H), _f32)
    v3 = pl.BlockSpec((1, 1, _RN), lambda i: (i, 0, 0))
    full = pl.BlockSpec((128, _H), lambda i: (0, 0))
    blkh = pl.BlockSpec((_RN, _H), lambda i: (i, 0))
    return pl.pallas_call(
        body, grid=(_GN,),
        in_specs=[v3, v3, v3, v3, full, full, full],
        out_specs=[blkh] * 4,
        out_shape=[sh] * 4, compiler_params=_cparams(),
    )(x0, x1, nmw, nmm, a0p, a1p, wm0)


def _embed_edges(e0, e1, emw, emm, b0p, b1p, wm0):
    def body(e0_r, e1_r, mw_r, mm_r, b0_r, b1_r, wm_r, ow_r, om_r, ouw_r, oum_r):
        i0 = e0_r[0, 0, :]
        i1 = e1_r[0, 0, :]
        h = _onehot_dot(i0, b0_r, 32) + _onehot_dot(i1, b1_r, 32)
        ew = h * (1.0 - mw_r[0, 0, :])[:, None]
        em = h * (1.0 - mm_r[0, 0, :])[:, None]
        ow_r[...] = ew
        om_r[...] = em
        ouw_r[...] = jnp.dot(ew, wm_r[...], preferred_element_type=_f32)
        oum_r[...] = jnp.dot(em, wm_r[...], preferred_element_type=_f32)

    sh = jax.ShapeDtypeStruct((_EP, _H), _f32)
    v3 = pl.BlockSpec((1, 1, _RE), lambda i: (i, 0, 0))
    full = pl.BlockSpec((32, _H), lambda i: (0, 0))
    wfull = pl.BlockSpec((_H, _H), lambda i: (0, 0))
    blk = pl.BlockSpec((_RE, _H), lambda i: (i, 0))
    return pl.pallas_call(
        body, grid=(_GE,),
        in_specs=[v3, v3, v3, v3, full, full, wfull],
        out_specs=[blk] * 4,
        out_shape=[sh] * 4, compiler_params=_cparams(),
    )(e0, e1, emw, emm, b0p, b1p, wm0)


def _node_update(h, parts, degp, w, wm_next, emit_z):
    def body(h_r, p0_r, p1_r, d0_r, d1_r, w_r, wm_r, o_r, *outs):
        agg = p0_r[...] + p1_r[...]
        deg = d0_r[:, 0:1] + d1_r[:, 0:1]
        inv = 1.0 / jnp.maximum(deg, 1.0)
        hn = jnp.maximum(
            jnp.dot(h_r[...], w_r[...], preferred_element_type=_f32) + agg * inv, 0.0)
        o_r[...] = hn
        if emit_z:
            outs[0][...] = jnp.dot(hn, wm_r[...], preferred_element_type=_f32)

    blk = pl.BlockSpec((_RN, _H), lambda i: (i, 0))
    p0 = pl.BlockSpec((_RN, _H), lambda i: (i, 0))
    p1 = pl.BlockSpec((_RN, _H), lambda i: (i + _GN, 0))
    d0 = pl.BlockSpec((_RN, _H), lambda i: (i, 0))
    d1 = pl.BlockSpec((_RN, _H), lambda i: (i + _GN, 0))
    full = pl.BlockSpec((_H, _H), lambda i: (0, 0))
    sh = jax.ShapeDtypeStruct((_NP, _H), _f32)
    out_specs = [blk] + ([blk] if emit_z else [])
    out_shape = [sh] + ([sh] if emit_z else [])
    return pl.pallas_call(
        body, grid=(_GN,), in_specs=[blk, p0, p1, d0, d1, full, full],
        out_specs=out_specs, out_shape=out_shape,
        compiler_params=_cparams(),
    )(h, parts, parts, degp, degp, w, wm_next)


def _edge_update(ea, gc, w, wm_next, emit_u):
    def body(ea_r, ga_r, gb_r, w_r, wm_r, o_r, *outs):
        en = jnp.maximum(
            jnp.dot(ea_r[...], w_r[...], preferred_element_type=_f32)
            + ga_r[...] + gb_r[...], 0.0)
        o_r[...] = en
        if emit_u:
            outs[0][...] = jnp.dot(en, wm_r[...], preferred_element_type=_f32)

    blk = pl.BlockSpec((_RE, _H), lambda i: (i, 0))
    blka = pl.BlockSpec((_RE, _H), lambda i: (i, 0))
    blkb = pl.BlockSpec((_RE, _H), lambda i: (i + _GE, 0))
    full = pl.BlockSpec((_H, _H), lambda i: (0, 0))
    sh = jax.ShapeDtypeStruct((_EP, _H), _f32)
    out_specs = [blk] + ([blk] if emit_u else [])
    out_shape = [sh] + ([sh] if emit_u else [])
    return pl.pallas_call(
        body, grid=(_GE,), in_specs=[blk, blka, blkb, full, full],
        out_specs=out_specs, out_shape=out_shape, compiler_params=_cparams(),
    )(ea, gc, gc, w, wm_next)


def _acc2(ref, val, i):
    @pl.when(i == 0)
    def _():
        ref[...] = jnp.zeros((1, 1), _f32)

    ref[...] += jnp.reshape(val, (1, 1))


def _ce_from_logits(z, lab, ncls_real):
    cm = lax.broadcasted_iota(_i32, z.shape, 1) < ncls_real
    z = jnp.where(cm, z, -1e30)
    m = jnp.max(z, axis=1, keepdims=True)
    lse = m + jnp.log(jnp.sum(jnp.exp(z - m), axis=1, keepdims=True))
    oh = lab[:, None] == lax.broadcasted_iota(_i32, z.shape, 1)
    zy = jnp.sum(jnp.where(oh, z, 0.0), axis=1, keepdims=True)
    return lse - zy


def _node_ce(x1, x2, x3, wn3, bn, lab3, w3):
    def body(x1_r, x2_r, x3_r, w_r, b_r, lab_r, wt_r, s_r, sw_r):
        i = pl.program_id(0)
        z = (jnp.dot(x1_r[...], w_r[0], preferred_element_type=_f32)
             + jnp.dot(x2_r[...], w_r[1], preferred_element_type=_f32)
             + jnp.dot(x3_r[...], w_r[2], preferred_element_type=_f32)
             + b_r[...])
        nll = _ce_from_logits(z, lab_r[0, 0, :], _NA)
        wt = wt_r[0, 0, :][:, None]
        _acc2(s_r, jnp.sum(nll * wt), i)
        _acc2(sw_r, jnp.sum(wt), i)

    blk = pl.BlockSpec((_RN, _H), lambda i: (i, 0))
    wfull = pl.BlockSpec((3, _H, 128), lambda i: (0, 0, 0))
    bfull = pl.BlockSpec((1, 128), lambda i: (0, 0))
    v3 = pl.BlockSpec((1, 1, _RN), lambda i: (i, 0, 0))
    sc = pl.BlockSpec((1, 1), lambda i: (0, 0))
    s1 = jax.ShapeDtypeStruct((1, 1), _f32)
    return pl.pallas_call(
        body, grid=(_GN,), in_specs=[blk, blk, blk, wfull, bfull, v3, v3],
        out_specs=[sc, sc], out_shape=[s1, s1], compiler_params=_cparams(),
    )(x1, x2, x3, wn3, bn, lab3, w3)


def _edge_ce(es, gcs, we3, be, lab3, w3, emit_e):
    def body(e1_r, e2_r, e3_r, a1_r, a2_r, a3_r, b1_r, b2_r, b3_r,
             w_r, b_r, lab_r, wt_r, *outs):
        i = pl.program_id(0)
        c1 = e1_r[...] + a1_r[...] + b1_r[...]
        c2 = e2_r[...] + a2_r[...] + b2_r[...]
        c3 = e3_r[...] + a3_r[...] + b3_r[...]
        z = (jnp.dot(c1, w_r[0], preferred_element_type=_f32)
             + jnp.dot(c2, w_r[1], preferred_element_type=_f32)
             + jnp.dot(c3, w_r[2], preferred_element_type=_f32)
             + b_r[...])
        nll = _ce_from_logits(z, lab_r[0, 0, :], _NB)
        wt = wt_r[0, 0, :][:, None]
        _acc2(outs[0], jnp.sum(nll * wt), i)
        _acc2(outs[1], jnp.sum(wt), i)
        if emit_e:
            outs[2][...] = c1
            outs[3][...] = c2
            outs[4][...] = c3

    blk = pl.BlockSpec((_RE, _H), lambda i: (i, 0))
    blkb = pl.BlockSpec((_RE, _H), lambda i: (i + _GE, 0))
    wfull = pl.BlockSpec((3, _H, 32), lambda i: (0, 0, 0))
    bfull = pl.BlockSpec((1, 32), lambda i: (0, 0))
    v3 = pl.BlockSpec((1, 1, _RE), lambda i: (i, 0, 0))
    sc = pl.BlockSpec((1, 1), lambda i: (0, 0))
    s1 = jax.ShapeDtypeStruct((1, 1), _f32)
    eo = jax.ShapeDtypeStruct((_EP, _H), _f32)
    out_specs = [sc, sc] + ([blk] * 3 if emit_e else [])
    out_shape = [s1, s1] + ([eo] * 3 if emit_e else [])
    return pl.pallas_call(
        body, grid=(_GE,),
        in_specs=[blk] * 3 + [blk] * 3 + [blkb] * 3 + [wfull, bfull, v3, v3],
        out_specs=out_specs, out_shape=out_shape, compiler_params=_cparams(),
    )(es[0], es[1], es[2], gcs[0], gcs[1], gcs[2], gcs[0], gcs[1], gcs[2],
      we3, be, lab3, w3)


def _bce_elem(z, y):
    return jnp.maximum(z, 0.0) - z * y + jnp.log(1.0 + jnp.exp(-jnp.abs(z)))


def _node_comp(x1, x2, x3, wf0, bf0, wf1, bf1, ws0, bs0, ws1, bs1,
               labfg, labsc3, nmm13):
    def body(x1_r, x2_r, x3_r, wf0_r, bf0_r, wf1_r, bf1_r,
             ws0_r, bs0_r, ws1_r, bs1_r, yfg_r, ysc_r, m_r,
             sfg_r, ssc_r, sw_r):
        i = pl.program_id(0)
        w = (1.0 - m_r[0, 0, :])[:, None]

        hf = (jnp.dot(x1_r[...], wf0_r[0], preferred_element_type=_f32)
              + jnp.dot(x2_r[...], wf0_r[1], preferred_element_type=_f32)
              + jnp.dot(x3_r[...], wf0_r[2], preferred_element_type=_f32)
              + bf0_r[...])
        hf = jnp.maximum(hf, 0.0)
        zf = jnp.dot(hf, wf1_r[...], preferred_element_type=_f32) + bf1_r[...]
        cmf = (lax.broadcasted_iota(_i32, zf.shape, 1) < _FG).astype(_f32)
        perf = jnp.sum(_bce_elem(zf, yfg_r[...]) * cmf, axis=1, keepdims=True) / _FG

        hs = (jnp.dot(x1_r[...], ws0_r[0], preferred_element_type=_f32)
              + jnp.dot(x2_r[...], ws0_r[1], preferred_element_type=_f32)
              + jnp.dot(x3_r[...], ws0_r[2], preferred_element_type=_f32)
              + bs0_r[...])
        hs = jnp.maximum(hs, 0.0)
        zs = jnp.dot(hs, ws1_r[...], preferred_element_type=_f32) + bs1_r[...]
        ysc = ysc_r[0, 0, :][:, None]
        pers = _bce_elem(zs[:, 0:1], ysc)
        _acc2(sfg_r, jnp.sum(perf * w), i)
        _acc2(ssc_r, jnp.sum(pers * w), i)
        _acc2(sw_r, jnp.sum(w), i)

    blk = pl.BlockSpec((_RN, _H), lambda i: (i, 0))
    w0full = pl.BlockSpec((3, _H, 384), lambda i: (0, 0, 0))
    b0full = pl.BlockSpec((1, 384), lambda i: (0, 0))
    w1full = pl.BlockSpec((384, 128), lambda i: (0, 0))
    b1full = pl.BlockSpec((1, 128), lambda i: (0, 0))
    yfull = pl.BlockSpec((_RN, 128), lambda i: (i, 0))
    v3 = pl.BlockSpec((1, 1, _RN), lambda i: (i, 0, 0))
    sc = pl.BlockSpec((1, 1), lambda i: (0, 0))
    s1 = jax.ShapeDtypeStruct((1, 1), _f32)
    return pl.pallas_call(
        body, grid=(_GN,),
        in_specs=[blk, blk, blk, w0full, b0full, w1full, b1full,
                  w0full, b0full, w1full, b1full, yfull, v3, v3],
        out_specs=[sc, sc, sc], out_shape=[s1, s1, s1], compiler_params=_cparams(),
    )(x1, x2, x3, wf0, bf0, wf1, bf1, ws0, bs0, ws1, bs1, labfg, labsc3, nmm13)


def _edge_comp(e1, e2, e3, wb0, bb0, wb1, bb1, labbr3, emm13):
    def body(e1_r, e2_r, e3_r, w0_r, b0_r, w1_r, b1_r, y_r, m_r, s_r, sw_r):
        i = pl.program_id(0)
        w = (1.0 - m_r[0, 0, :])[:, None]
        hh = (jnp.dot(e1_r[...], w0_r[0], preferred_element_type=_f32)
              + jnp.dot(e2_r[...], w0_r[1], preferred_element_type=_f32)
              + jnp.dot(e3_r[...], w0_r[2], preferred_element_type=_f32)
              + b0_r[...])
        hh = jnp.maximum(hh, 0.0)
        z = jnp.dot(hh, w1_r[...], preferred_element_type=_f32) + b1_r[...]
        y = y_r[0, 0, :][:, None]
        per = _bce_elem(z[:, 0:1], y)
        _acc2(s_r, jnp.sum(per * w), i)
        _acc2(sw_r, jnp.sum(w), i)

    blk = pl.BlockSpec((_RE, _H), lambda i: (i, 0))
    w0full = pl.BlockSpec((3, _H, 384), lambda i: (0, 0, 0))
    b0full = pl.BlockSpec((1, 384), lambda i: (0, 0))
    w1full = pl.BlockSpec((384, 128), lambda i: (0, 0))
    b1full = pl.BlockSpec((1, 128), lambda i: (0, 0))
    v3 = pl.BlockSpec((1, 1, _RE), lambda i: (i, 0, 0))
    sc = pl.BlockSpec((1, 1), lambda i: (0, 0))
    s1 = jax.ShapeDtypeStruct((1, 1), _f32)
    return pl.pallas_call(
        body, grid=(_GE,),
        in_specs=[blk, blk, blk, w0full, b0full, w1full, b1full, v3, v3],
        out_specs=[sc, sc], out_shape=[s1, s1], compiler_params=_cparams(),
    )(e1, e2, e3, wb0, bb0, wb1, bb1, labbr3, emm13)


def _padn(v, fill=0):
    return jnp.pad(v, ((0, _NP - _N),) + ((0, 0),) * (v.ndim - 1), constant_values=fill)


def _pade(v, fill=0):
    return jnp.pad(v, ((0, _EP - _E),) + ((0, 0),) * (v.ndim - 1), constant_values=fill)


def _r3(v, r):
    return v.reshape(-1, 1, r)


def kernel(x, edge_attr, edge_index, node_mask, edge_mask, node_mask_motif,
           edge_mask_motif, label_fg, label_brics, label_scaffold, params):
    p = params

    x0 = _r3(_padn(x[:, 0].astype(_i32)), _RN)
    x1 = _r3(_padn(x[:, 1].astype(_i32)), _RN)
    e0 = _r3(_pade(edge_attr[:, 0].astype(_i32)), _RE)
    e1 = _r3(_pade(edge_attr[:, 1].astype(_i32)), _RE)
    src = _pade(edge_index[0].astype(_i32), 0)
    dst = _pade(edge_index[1].astype(_i32), _NP - 1)
    srcdst2 = jnp.concatenate([src.reshape(-1, _CH), dst.reshape(-1, _CH)])

    nmw = _r3(_padn(node_mask.astype(_f32)), _RN)
    nmm0 = _r3(_padn(node_mask_motif.astype(_f32)), _RN)
    nmm1 = _r3(_padn(node_mask_motif.astype(_f32), 1.0), _RN)
    emw = _r3(_pade(edge_mask.astype(_f32)), _RE)
    emm0 = _r3(_pade(edge_mask_motif.astype(_f32)), _RE)
    emm1 = _r3(_pade(edge_mask_motif.astype(_f32), 1.0), _RE)

    labn = x0
    labe = e0
    labfg = jnp.pad(_padn(label_fg), ((0, 0), (0, 128 - _FG)))
    labsc = _r3(_padn(label_scaffold[:, 0]), _RN)
    labbr = _r3(_pade(label_brics[:, 0]), _RE)

    a0p = jnp.pad(p['emb_a0'], ((0, 128 - _NA), (0, 0)))
    a1p = jnp.pad(p['emb_a1'], ((0, 128 - _NA), (0, 0)))
    b0p = jnp.pad(p['emb_b0'], ((0, 32 - _NB), (0, 0)))
    b1p = jnp.pad(p['emb_b1'], ((0, 32 - _NB), (0, 0)))

    wn3 = jnp.pad(p['Wn'], ((0, 0), (0, 128 - _NA))).reshape(3, _H, 128)
    bn = jnp.pad(p['bn'], (0, 128 - _NA)).reshape(1, 128)
    we3 = jnp.pad(p['We'], ((0, 0), (0, 32 - _NB))).reshape(3, _H, 32)
    be = jnp.pad(p['be'], (0, 32 - _NB)).reshape(1, 32)

    wf0 = p['W_fg_0'].reshape(3, _H, 384)
    bf0 = p['b_fg_0'].reshape(1, 384)
    wf1 = jnp.pad(p['W_fg_1'], ((0, 0), (0, 128 - _FG)))
    bf1 = jnp.pad(p['b_fg_1'], (0, 128 - _FG)).reshape(1, 128)
    ws0 = p['W_scaffold_0'].reshape(3, _H, 384)
    bs0 = p['b_scaffold_0'].reshape(1, 384)
    ws1 = jnp.pad(p['W_scaffold_1'], ((0, 0), (0, 127)))
    bs1 = jnp.pad(p['b_scaffold_1'], (0, 127)).reshape(1, 128)
    wb0 = p['W_brics_0'].reshape(3, _H, 384)
    bb0 = p['b_brics_0'].reshape(1, 384)
    wb1 = jnp.pad(p['W_brics_1'], ((0, 0), (0, 127)))
    bb1 = jnp.pad(p['b_brics_1'], (0, 127)).reshape(1, 128)

    zeros_h = jnp.zeros((_CH, _H), _f32)

    wm = [p['W_msg%d' % i] for i in range(_L)]
    hw, hm, zw, zm = _embed_nodes(x0, x1, nmw, nmm0, a0p, a1p, wm[0])
    eaw, eam, uw, um = _embed_edges(e0, e1, emw, emm0, b0p, b1p, wm[0])
    degp = _sc_scatter_add(jnp.ones((_EP, _H), _f32), dst, zeros_h)

    def run_pass(h, z, ea, u, emit_e, nmask, emask):
        xs, gcs, es = [], [], []
        for i in range(_L):
            emit = i < _L - 1
            wm_next = wm[i + 1] if emit else wm[i]
            parts = _sc_gather_scatter(z, u, src, dst, zeros_h)
            nouts = _node_update(h, parts, degp, p['W_self%d' % i], wm_next, emit)
            h = nouts[0]
            gc = _sc_gather(h, srcdst2)
            uouts = _edge_update(ea, gc, p['W_edge%d' % i], wm_next, emit)
            ea = uouts[0]
            if emit:
                z, u = nouts[1], uouts[1]
            xs.append(h); es.append(ea); gcs.append(gc)
        sn, swn = _node_ce(xs[0], xs[1], xs[2], wn3, bn, labn, nmask)
        eouts = _edge_ce(es, gcs, we3, be, labe, emask, emit_e)
        return xs, eouts, sn, swn

    xs_w, eo_w, sn_w, swn_w = run_pass(hw, zw, eaw, uw, False, nmw, emw)
    se_w, swe_w = eo_w[0], eo_w[1]
    xs_m, eo_m, sn_m, swn_m = run_pass(hm, zm, eam, um, True, nmm0, emm0)
    se_m, swe_m = eo_m[0], eo_m[1]
    ec1, ec2, ec3 = eo_m[2], eo_m[3], eo_m[4]

    sfg, ssc, swc = _node_comp(xs_m[0], xs_m[1], xs_m[2], wf0, bf0, wf1, bf1,
                               ws0, bs0, ws1, bs1, labfg, labsc, nmm1)
    sbr, swbr = _edge_comp(ec1, ec2, ec3, wb0, bb0, wb1, bb1, labbr, emm1)

    def _div(a, b):
        return (a[0, 0] / jnp.maximum(b[0, 0], 1.0)).astype(_f32)

    ln_w = _div(sn_w, swn_w)
    le_w = _div(se_w, swe_w)
    ln_m = _div(sn_m, swn_m)
    le_m = _div(se_m, swe_m)
    l_fg = _div(sfg, swc)
    l_sc = _div(ssc, swc)
    l_br = _div(sbr, swbr)

    sep = jnp.stack([ln_w, le_w, ln_m, le_m, l_fg, l_br, l_sc])
    loss = ln_w + le_w + ln_m + le_m + l_fg + l_br + l_sc
    return (loss, sep)


# speedup vs baseline: 1.1430x; 1.0154x over previous
import functools

import jax
import jax.numpy as jnp
from jax import lax
from jax.experimental import pallas as pl
from jax.experimental.pallas import tpu as pltpu
from jax.experimental.pallas import tpu_sc as plsc

_N = 10000
_E = 160000
_H = 128
_L = 3
_NA = 119
_NB = 22
_FG = 39

_NP = 10240
_EP = 163840
_NC = 2
_NS = 16
_NW = _NC * _NS
_CH = 128
_PW = _EP // _NW
_NCHUNK = _PW // _CH
_RPT = _NP // _NS
_SLAB = _RPT // _CH
_BC = 4

_RN = 512
_RE = 512
_GN = _NP // _RN
_GE = _EP // _RE

_f32 = jnp.float32
_i32 = jnp.int32


def _mesh():
    return plsc.VectorSubcoreMesh(core_axis_name="c", subcore_axis_name="s")


def _sc_gather(table, idx2):
    D = table.shape[1]
    dt = table.dtype
    R = idx2.shape[0]
    rpw = R // _NW
    nb = 4

    @functools.partial(
        pl.kernel,
        mesh=_mesh(),
        out_type=jax.ShapeDtypeStruct((R * _CH, D), dt),
        scratch_types=[pltpu.VMEM((rpw, _CH), _i32)]
        + [pltpu.VMEM((_CH, D), dt)] * nb
        + [pltpu.SemaphoreType.DMA, pltpu.SemaphoreType.DMA],
    )
    def k(table_hbm, idx_hbm, out_hbm, idx_all, *rest):
        rows = list(rest[:nb])
        sem_g, sem_w = rest[nb], rest[nb + 1]
        wid = lax.axis_index("s") * _NC + lax.axis_index("c")
        irow0 = wid * rpw
        base = pl.multiple_of(wid * rpw * _CH, 8)
        pltpu.sync_copy(idx_hbm.at[pl.ds(irow0, rpw)], idx_all)

        def g_start(j, b):
            pltpu.async_copy(table_hbm.at[idx_all.at[j]], rows[b], sem_g)

        def g_wait(j, b):
            pltpu.make_async_copy(table_hbm.at[idx_all.at[j]], rows[b], sem_g).wait()

        def w_start(j, b):
            off = pl.multiple_of(base + j * _CH, 8)
            pltpu.async_copy(rows[b], out_hbm.at[pl.ds(off, _CH)], sem_w)

        def w_wait(j, b):
            off = pl.multiple_of(base + j * _CH, 8)
            pltpu.make_async_copy(rows[b], out_hbm.at[pl.ds(off, _CH)], sem_w).wait()

        for b in range(nb):
            g_start(b, b)

        def body(t, carry):
            j0 = t * nb
            for b in range(nb):
                g_wait(j0 + b, b)
                w_start(j0 + b, b)
            for b in range(nb):
                w_wait(j0 + b, b)
                g_start(j0 + nb + b, b)
            return carry

        lax.fori_loop(0, rpw // nb - 1, body, 0)
        jl = rpw - nb
        for b in range(nb):
            g_wait(jl + b, b)
            w_start(jl + b, b)
        for b in range(nb):
            w_wait(jl + b, b)

    return k(table, idx2)


def _sc_scatter_add(vals, idx, zeros_h):
    D = vals.shape[1]

    @functools.partial(
        pl.kernel,
        mesh=_mesh(),
        out_type=jax.ShapeDtypeStruct((2 * _NP, D), _f32),
        scratch_types=[
            pltpu.VMEM((_CH,), _i32),
            pltpu.VMEM((_CH,), _i32),
            pltpu.VMEM((_CH, D), _f32),
            pltpu.VMEM((_CH, D), _f32),
            pltpu.VMEM_SHARED((_NP, D), _f32),
            pltpu.SemaphoreType.DMA,
            pltpu.SemaphoreType.DMA,
            pltpu.SemaphoreType.DMA,
        ],
    )
    def k(vals_hbm, idx_hbm, zeros_hbm, out_hbm,
          i0, i1, v0, v1, acc_sh, sem_i, sem_l, sem_s):
        cid = lax.axis_index("c")
        sid = lax.axis_index("s")
        wid = sid * _NC + cid
        r0 = sid * _RPT
        idxs = [i0, i1]
        bufs = [v0, v1]
        nb = 2
        pltpu.sync_copy(zeros_hbm, v0)

        def zbody(kk, carry):
            pltpu.sync_copy(v0, acc_sh.at[pl.ds(pl.multiple_of(r0 + kk * _CH, 8), _CH)])
            return carry

        lax.fori_loop(0, _SLAB, zbody, 0)
        plsc.subcore_barrier()

        base = pl.multiple_of(wid * _PW, 8)

        def l_start(j, b):
            off = pl.multiple_of(base + j * _CH, 8)
            pltpu.async_copy(idx_hbm.at[pl.ds(off, _CH)], idxs[b], sem_i)
            pltpu.async_copy(vals_hbm.at[pl.ds(off, _CH)], bufs[b], sem_l)

        def l_wait(j, b):
            off = pl.multiple_of(base + j * _CH, 8)
            pltpu.make_async_copy(idx_hbm.at[pl.ds(off, _CH)], idxs[b], sem_i).wait()
            pltpu.make_async_copy(vals_hbm.at[pl.ds(off, _CH)], bufs[b], sem_l).wait()

        def s_start(b):
            pltpu.async_copy(bufs[b], acc_sh.at[idxs[b]], sem_s, add=True)

        def s_wait(b):
            pltpu.make_async_copy(bufs[b], acc_sh.at[idxs[b]], sem_s).wait()

        for b in range(nb):
            l_start(b, b)

        def body(t, carry):
            j0 = t * nb
            for b in range(nb):
                l_wait(j0 + b, b)
                s_start(b)
            for b in range(nb):
                s_wait(b)
                l_start(j0 + nb + b, b)
            return carry

        lax.fori_loop(0, _NCHUNK // nb - 1, body, 0)
        jl = _NCHUNK - nb
        for b in range(nb):
            l_wait(jl + b, b)
            s_start(b)
        for b in range(nb):
            s_wait(b)
        plsc.subcore_barrier()

        def obody(kk, carry):
            src_off = pl.multiple_of(r0 + kk * _CH, 8)
            dst_off = pl.multiple_of(cid * _NP + r0 + kk * _CH, 8)
            pltpu.sync_copy(acc_sh.at[pl.ds(src_off, _CH)], v0)
            pltpu.sync_copy(v0, out_hbm.at[pl.ds(dst_off, _CH)])
            return carry

        lax.fori_loop(0, _SLAB, obody, 0)

    return k(vals, idx, zeros_h)


def _sc_gather_scatter(z, u, src64, dst64, zeros_h):
    D = z.shape[1]
    ch = 64
    nchunk = _PW // ch

    @functools.partial(
        pl.kernel,
        mesh=_mesh(),
        out_type=jax.ShapeDtypeStruct((2 * _NP, D), _f32),
        scratch_types=[
            pltpu.VMEM((nchunk // 2, ch), _i32),
            pltpu.VMEM((nchunk // 2, ch), _i32),
            pltpu.VMEM((ch, D), _f32),
            pltpu.VMEM((ch, D), _f32),
            pltpu.VMEM((ch, D), _f32),
            pltpu.VMEM((ch, D), _f32),
            pltpu.VMEM_SHARED((_NP, D), _f32),
            pltpu.SemaphoreType.DMA,
            pltpu.SemaphoreType.DMA,
            pltpu.SemaphoreType.DMA,
        ],
    )
    def k(z_hbm, u_hbm, src_hbm, dst_hbm, zeros_hbm, out_hbm,
          sidx, didx, g0, g1, u0, u1, acc_sh, sem_g, sem_l, sem_s):
        cid = lax.axis_index("c")
        sid = lax.axis_index("s")
        wid = sid * _NC + cid
        r0 = sid * _RPT
        gbufs = [g0, g1]
        ubufs = [u0, u1]
        nb = 2
        irow0 = wid * nchunk
        pltpu.sync_copy(zeros_hbm.at[pl.ds(0, ch)], g0)

        def zbody(kk, carry):
            pltpu.sync_copy(g0, acc_sh.at[pl.ds(pl.multiple_of(r0 + kk * ch, 8), ch)])
            return carry

        lax.fori_loop(0, _RPT // ch, zbody, 0)
        plsc.subcore_barrier()

        base = pl.multiple_of(wid * _PW, 8)
        nh = nchunk // 2

        for ph in range(2):
            jb = ph * nh
            pltpu.sync_copy(src_hbm.at[pl.ds(irow0 + jb, nh)], sidx)
            pltpu.sync_copy(dst_hbm.at[pl.ds(irow0 + jb, nh)], didx)

            def l_start(j, b, jb=jb):
                off = pl.multiple_of(base + (jb + j) * ch, 8)
                pltpu.async_copy(z_hbm.at[sidx.at[j]], gbufs[b], sem_g)
                pltpu.async_copy(u_hbm.at[pl.ds(off, ch)], ubufs[b], sem_l)

            def l_wait(j, b, jb=jb):
                off = pl.multiple_of(base + (jb + j) * ch, 8)
                pltpu.make_async_copy(z_hbm.at[sidx.at[j]], gbufs[b], sem_g).wait()
                pltpu.make_async_copy(u_hbm.at[pl.ds(off, ch)], ubufs[b], sem_l).wait()

            def s_start(j, b):
                pltpu.async_copy(gbufs[b], acc_sh.at[didx.at[j]], sem_s, add=True)
                pltpu.async_copy(ubufs[b], acc_sh.at[didx.at[j]], sem_s, add=True)

            def s_wait(j, b):
                pltpu.make_async_copy(gbufs[b], acc_sh.at[didx.at[j]], sem_s).wait()
                pltpu.make_async_copy(ubufs[b], acc_sh.at[didx.at[j]], sem_s).wait()

            for b in range(nb):
                l_start(b, b)

            def body(t, carry, l_start=l_start, l_wait=l_wait,
                     s_start=s_start, s_wait=s_wait):
                j0 = t * nb
                for b in range(nb):
                    l_wait(j0 + b, b)
                    s_start(j0 + b, b)
                for b in range(nb):
                    s_wait(j0 + b, b)
                    l_start(j0 + nb + b, b)
                return carry

            lax.fori_loop(0, nh // nb - 1, body, 0)
            jl = nh - nb
            for b in range(nb):
                l_wait(jl + b, b)
                s_start(jl + b, b)
            for b in range(nb):
                s_wait(jl + b, b)
        plsc.subcore_barrier()

        def obody(kk, carry):
            src_off = pl.multiple_of(r0 + kk * ch, 8)
            dst_off = pl.multiple_of(cid * _NP + r0 + kk * ch, 8)
            pltpu.sync_copy(acc_sh.at[pl.ds(src_off, ch)], g0)
            pltpu.sync_copy(g0, out_hbm.at[pl.ds(dst_off, ch)])
            return carry

        lax.fori_loop(0, _RPT // ch, obody, 0)

    return k(z, u, src64, dst64, zeros_h)


def _cparams():
    return pltpu.CompilerParams(dimension_semantics=("arbitrary",))


def _onehot_dot(idx, table_ref, ncls):
    oh = (idx[:, None] == lax.broadcasted_iota(_i32, (idx.shape[0], ncls), 1)).astype(_f32)
    return jax.lax.dot_general(oh, table_ref[...], (((1,), (0,)), ((), ())),
                               precision=jax.lax.Precision.HIGHEST,
                               preferred_element_type=_f32)


def _embed_nodes(x0, x1, nmw, nmm, a0p, a1p, wm0):
    def body(x0_r, x1_r, mw_r, mm_r, a0_r, a1_r, wm_r, ow_r, om_r, ozw_r, ozm_r):
        i0 = x0_r[0, 0, :]
        i1 = x1_r[0, 0, :]
        h = _onehot_dot(i0, a0_r, 128) + _onehot_dot(i1, a1_r, 128)
        hw = h * (1.0 - mw_r[0, 0, :])[:, None]
        hm = h * (1.0 - mm_r[0, 0, :])[:, None]
        ow_r[...] = hw
        om_r[...] = hm
        ozw_r[...] = jnp.dot(hw, wm_r[...], preferred_element_type=_f32)
        ozm_r[...] = jnp.dot(hm, wm_r[...], preferred_element_type=_f32)

    sh = jax.ShapeDtypeStruct((_NP, _H), _f32)
    v3 = pl.BlockSpec((1, 1, _RN), lambda i: (i, 0, 0))
    full = pl.BlockSpec((128, _H), lambda i: (0, 0))
    blkh = pl.BlockSpec((_RN, _H), lambda i: (i, 0))
    return pl.pallas_call(
        body, grid=(_GN,),
        in_specs=[v3, v3, v3, v3, full, full, full],
        out_specs=[blkh] * 4,
        out_shape=[sh] * 4, compiler_params=_cparams(),
    )(x0, x1, nmw, nmm, a0p, a1p, wm0)


def _embed_edges(e0, e1, emw, emm, b0p, b1p, wm0):
    def body(e0_r, e1_r, mw_r, mm_r, b0_r, b1_r, wm_r, ow_r, om_r, ouw_r, oum_r):
        i0 = e0_r[0, 0, :]
        i1 = e1_r[0, 0, :]
        h = _onehot_dot(i0, b0_r, 32) + _onehot_dot(i1, b1_r, 32)
        ew = h * (1.0 - mw_r[0, 0, :])[:, None]
        em = h * (1.0 - mm_r[0, 0, :])[:, None]
        ow_r[...] = ew
        om_r[...] = em
        ouw_r[...] = jnp.dot(ew, wm_r[...], preferred_element_type=_f32)
        oum_r[...] = jnp.dot(em, wm_r[...], preferred_element_type=_f32)

    sh = jax.ShapeDtypeStruct((_EP, _H), _f32)
    v3 = pl.BlockSpec((1, 1, _RE), lambda i: (i, 0, 0))
    full = pl.BlockSpec((32, _H), lambda i: (0, 0))
    wfull = pl.BlockSpec((_H, _H), lambda i: (0, 0))
    blk = pl.BlockSpec((_RE, _H), lambda i: (i, 0))
    return pl.pallas_call(
        body, grid=(_GE,),
        in_specs=[v3, v3, v3, v3, full, full, wfull],
        out_specs=[blk] * 4,
        out_shape=[sh] * 4, compiler_params=_cparams(),
    )(e0, e1, emw, emm, b0p, b1p, wm0)


def _node_update(h, parts, degp, w, wm_next, emit_z):
    def body(h_r, p0_r, p1_r, d0_r, d1_r, w_r, wm_r, o_r, *outs):
        agg = p0_r[...] + p1_r[...]
        deg = d0_r[:, 0:1] + d1_r[:, 0:1]
        inv = 1.0 / jnp.maximum(deg, 1.0)
        hn = jnp.maximum(
            jnp.dot(h_r[...], w_r[...], preferred_element_type=_f32) + agg * inv, 0.0)
        o_r[...] = hn
        if emit_z:
            outs[0][...] = jnp.dot(hn, wm_r[...], preferred_element_type=_f32)

    blk = pl.BlockSpec((_RN, _H), lambda i: (i, 0))
    p0 = pl.BlockSpec((_RN, _H), lambda i: (i, 0))
    p1 = pl.BlockSpec((_RN, _H), lambda i: (i + _GN, 0))
    d0 = pl.BlockSpec((_RN, _H), lambda i: (i, 0))
    d1 = pl.BlockSpec((_RN, _H), lambda i: (i + _GN, 0))
    full = pl.BlockSpec((_H, _H), lambda i: (0, 0))
    sh = jax.ShapeDtypeStruct((_NP, _H), _f32)
    out_specs = [blk] + ([blk] if emit_z else [])
    out_shape = [sh] + ([sh] if emit_z else [])
    return pl.pallas_call(
        body, grid=(_GN,), in_specs=[blk, p0, p1, d0, d1, full, full],
        out_specs=out_specs, out_shape=out_shape,
        compiler_params=_cparams(),
    )(h, parts, parts, degp, degp, w, wm_next)


def _edge_update(ea, gc, w, wm_next, emit_u):
    def body(ea_r, ga_r, gb_r, w_r, wm_r, o_r, *outs):
        en = jnp.maximum(
            jnp.dot(ea_r[...], w_r[...], preferred_element_type=_f32)
            + ga_r[...] + gb_r[...], 0.0)
        o_r[...] = en
        if emit_u:
            outs[0][...] = jnp.dot(en, wm_r[...], preferred_element_type=_f32)

    blk = pl.BlockSpec((_RE, _H), lambda i: (i, 0))
    blka = pl.BlockSpec((_RE, _H), lambda i: (i, 0))
    blkb = pl.BlockSpec((_RE, _H), lambda i: (i + _GE, 0))
    full = pl.BlockSpec((_H, _H), lambda i: (0, 0))
    sh = jax.ShapeDtypeStruct((_EP, _H), _f32)
    out_specs = [blk] + ([blk] if emit_u else [])
    out_shape = [sh] + ([sh] if emit_u else [])
    return pl.pallas_call(
        body, grid=(_GE,), in_specs=[blk, blka, blkb, full, full],
        out_specs=out_specs, out_shape=out_shape, compiler_params=_cparams(),
    )(ea, gc, gc, w, wm_next)


def _acc2(ref, val, i):
    @pl.when(i == 0)
    def _():
        ref[...] = jnp.zeros((1, 1), _f32)

    ref[...] += jnp.reshape(val, (1, 1))


def _ce_from_logits(z, lab, ncls_real):
    cm = lax.broadcasted_iota(_i32, z.shape, 1) < ncls_real
    z = jnp.where(cm, z, -1e30)
    m = jnp.max(z, axis=1, keepdims=True)
    lse = m + jnp.log(jnp.sum(jnp.exp(z - m), axis=1, keepdims=True))
    oh = lab[:, None] == lax.broadcasted_iota(_i32, z.shape, 1)
    zy = jnp.sum(jnp.where(oh, z, 0.0), axis=1, keepdims=True)
    return lse - zy


def _node_ce(x1, x2, x3, wn3, bn, lab3, w3):
    def body(x1_r, x2_r, x3_r, w_r, b_r, lab_r, wt_r, s_r, sw_r):
        i = pl.program_id(0)
        z = (jnp.dot(x1_r[...], w_r[0], preferred_element_type=_f32)
             + jnp.dot(x2_r[...], w_r[1], preferred_element_type=_f32)
             + jnp.dot(x3_r[...], w_r[2], preferred_element_type=_f32)
             + b_r[...])
        nll = _ce_from_logits(z, lab_r[0, 0, :], _NA)
        wt = wt_r[0, 0, :][:, None]
        _acc2(s_r, jnp.sum(nll * wt), i)
        _acc2(sw_r, jnp.sum(wt), i)

    blk = pl.BlockSpec((_RN, _H), lambda i: (i, 0))
    wfull = pl.BlockSpec((3, _H, 128), lambda i: (0, 0, 0))
    bfull = pl.BlockSpec((1, 128), lambda i: (0, 0))
    v3 = pl.BlockSpec((1, 1, _RN), lambda i: (i, 0, 0))
    sc = pl.BlockSpec((1, 1), lambda i: (0, 0))
    s1 = jax.ShapeDtypeStruct((1, 1), _f32)
    return pl.pallas_call(
        body, grid=(_GN,), in_specs=[blk, blk, blk, wfull, bfull, v3, v3],
        out_specs=[sc, sc], out_shape=[s1, s1], compiler_params=_cparams(),
    )(x1, x2, x3, wn3, bn, lab3, w3)


def _edge_ce(es, gcs, we3, be, lab3, w3, emit_e):
    def body(e1_r, e2_r, e3_r, a1_r, a2_r, a3_r, b1_r, b2_r, b3_r,
             w_r, b_r, lab_r, wt_r, *outs):
        i = pl.program_id(0)
        c1 = e1_r[...] + a1_r[...] + b1_r[...]
        c2 = e2_r[...] + a2_r[...] + b2_r[...]
        c3 = e3_r[...] + a3_r[...] + b3_r[...]
        z = (jnp.dot(c1, w_r[0], preferred_element_type=_f32)
             + jnp.dot(c2, w_r[1], preferred_element_type=_f32)
             + jnp.dot(c3, w_r[2], preferred_element_type=_f32)
             + b_r[...])
        nll = _ce_from_logits(z, lab_r[0, 0, :], _NB)
        wt = wt_r[0, 0, :][:, None]
        _acc2(outs[0], jnp.sum(nll * wt), i)
        _acc2(outs[1], jnp.sum(wt), i)
        if emit_e:
            outs[2][...] = c1
            outs[3][...] = c2
            outs[4][...] = c3

    blk = pl.BlockSpec((_RE, _H), lambda i: (i, 0))
    blkb = pl.BlockSpec((_RE, _H), lambda i: (i + _GE, 0))
    wfull = pl.BlockSpec((3, _H, 32), lambda i: (0, 0, 0))
    bfull = pl.BlockSpec((1, 32), lambda i: (0, 0))
    v3 = pl.BlockSpec((1, 1, _RE), lambda i: (i, 0, 0))
    sc = pl.BlockSpec((1, 1), lambda i: (0, 0))
    s1 = jax.ShapeDtypeStruct((1, 1), _f32)
    eo = jax.ShapeDtypeStruct((_EP, _H), _f32)
    out_specs = [sc, sc] + ([blk] * 3 if emit_e else [])
    out_shape = [s1, s1] + ([eo] * 3 if emit_e else [])
    return pl.pallas_call(
        body, grid=(_GE,),
        in_specs=[blk] * 3 + [blk] * 3 + [blkb] * 3 + [wfull, bfull, v3, v3],
        out_specs=out_specs, out_shape=out_shape, compiler_params=_cparams(),
    )(es[0], es[1], es[2], gcs[0], gcs[1], gcs[2], gcs[0], gcs[1], gcs[2],
      we3, be, lab3, w3)


def _bce_elem(z, y):
    return jnp.maximum(z, 0.0) - z * y + jnp.log(1.0 + jnp.exp(-jnp.abs(z)))


def _node_comp(x1, x2, x3, wf0, bf0, wf1, bf1, ws0, bs0, ws1, bs1,
               labfg, labsc3, nmm13):
    def body(x1_r, x2_r, x3_r, wf0_r, bf0_r, wf1_r, bf1_r,
             ws0_r, bs0_r, ws1_r, bs1_r, yfg_r, ysc_r, m_r,
             sfg_r, ssc_r, sw_r):
        i = pl.program_id(0)
        w = (1.0 - m_r[0, 0, :])[:, None]

        hf = (jnp.dot(x1_r[...], wf0_r[0], preferred_element_type=_f32)
              + jnp.dot(x2_r[...], wf0_r[1], preferred_element_type=_f32)
              + jnp.dot(x3_r[...], wf0_r[2], preferred_element_type=_f32)
              + bf0_r[...])
        hf = jnp.maximum(hf, 0.0)
        zf = jnp.dot(hf, wf1_r[...], preferred_element_type=_f32) + bf1_r[...]
        cmf = (lax.broadcasted_iota(_i32, zf.shape, 1) < _FG).astype(_f32)
        perf = jnp.sum(_bce_elem(zf, yfg_r[...]) * cmf, axis=1, keepdims=True) / _FG

        hs = (jnp.dot(x1_r[...], ws0_r[0], preferred_element_type=_f32)
              + jnp.dot(x2_r[...], ws0_r[1], preferred_element_type=_f32)
              + jnp.dot(x3_r[...], ws0_r[2], preferred_element_type=_f32)
              + bs0_r[...])
        hs = jnp.maximum(hs, 0.0)
        zs = jnp.dot(hs, ws1_r[...], preferred_element_type=_f32) + bs1_r[...]
        ysc = ysc_r[0, 0, :][:, None]
        pers = _bce_elem(zs[:, 0:1], ysc)
        _acc2(sfg_r, jnp.sum(perf * w), i)
        _acc2(ssc_r, jnp.sum(pers * w), i)
        _acc2(sw_r, jnp.sum(w), i)

    blk = pl.BlockSpec((_RN, _H), lambda i: (i, 0))
    w0full = pl.BlockSpec((3, _H, 384), lambda i: (0, 0, 0))
    b0full = pl.BlockSpec((1, 384), lambda i: (0, 0))
    w1full = pl.BlockSpec((384, 128), lambda i: (0, 0))
    b1full = pl.BlockSpec((1, 128), lambda i: (0, 0))
    yfull = pl.BlockSpec((_RN, 128), lambda i: (i, 0))
    v3 = pl.BlockSpec((1, 1, _RN), lambda i: (i, 0, 0))
    sc = pl.BlockSpec((1, 1), lambda i: (0, 0))
    s1 = jax.ShapeDtypeStruct((1, 1), _f32)
    return pl.pallas_call(
        body, grid=(_GN,),
        in_specs=[blk, blk, blk, w0full, b0full, w1full, b1full,
                  w0full, b0full, w1full, b1full, yfull, v3, v3],
        out_specs=[sc, sc, sc], out_shape=[s1, s1, s1], compiler_params=_cparams(),
    )(x1, x2, x3, wf0, bf0, wf1, bf1, ws0, bs0, ws1, bs1, labfg, labsc3, nmm13)


def _edge_comp(e1, e2, e3, wb0, bb0, wb1, bb1, labbr3, emm13):
    def body(e1_r, e2_r, e3_r, w0_r, b0_r, w1_r, b1_r, y_r, m_r, s_r, sw_r):
        i = pl.program_id(0)
        w = (1.0 - m_r[0, 0, :])[:, None]
        hh = (jnp.dot(e1_r[...], w0_r[0], preferred_element_type=_f32)
              + jnp.dot(e2_r[...], w0_r[1], preferred_element_type=_f32)
              + jnp.dot(e3_r[...], w0_r[2], preferred_element_type=_f32)
              + b0_r[...])
        hh = jnp.maximum(hh, 0.0)
        z = jnp.dot(hh, w1_r[...], preferred_element_type=_f32) + b1_r[...]
        y = y_r[0, 0, :][:, None]
        per = _bce_elem(z[:, 0:1], y)
        _acc2(s_r, jnp.sum(per * w), i)
        _acc2(sw_r, jnp.sum(w), i)

    blk = pl.BlockSpec((_RE, _H), lambda i: (i, 0))
    w0full = pl.BlockSpec((3, _H, 384), lambda i: (0, 0, 0))
    b0full = pl.BlockSpec((1, 384), lambda i: (0, 0))
    w1full = pl.BlockSpec((384, 128), lambda i: (0, 0))
    b1full = pl.BlockSpec((1, 128), lambda i: (0, 0))
    v3 = pl.BlockSpec((1, 1, _RE), lambda i: (i, 0, 0))
    sc = pl.BlockSpec((1, 1), lambda i: (0, 0))
    s1 = jax.ShapeDtypeStruct((1, 1), _f32)
    return pl.pallas_call(
        body, grid=(_GE,),
        in_specs=[blk, blk, blk, w0full, b0full, w1full, b1full, v3, v3],
        out_specs=[sc, sc], out_shape=[s1, s1], compiler_params=_cparams(),
    )(e1, e2, e3, wb0, bb0, wb1, bb1, labbr3, emm13)


def _padn(v, fill=0):
    return jnp.pad(v, ((0, _NP - _N),) + ((0, 0),) * (v.ndim - 1), constant_values=fill)


def _pade(v, fill=0):
    return jnp.pad(v, ((0, _EP - _E),) + ((0, 0),) * (v.ndim - 1), constant_values=fill)


def _r3(v, r):
    return v.reshape(-1, 1, r)


def kernel(x, edge_attr, edge_index, node_mask, edge_mask, node_mask_motif,
           edge_mask_motif, label_fg, label_brics, label_scaffold, params):
    p = params

    x0 = _r3(_padn(x[:, 0].astype(_i32)), _RN)
    x1 = _r3(_padn(x[:, 1].astype(_i32)), _RN)
    e0 = _r3(_pade(edge_attr[:, 0].astype(_i32)), _RE)
    e1 = _r3(_pade(edge_attr[:, 1].astype(_i32)), _RE)
    src = _pade(edge_index[0].astype(_i32), 0)
    dst = _pade(edge_index[1].astype(_i32), _NP - 1)
    srcdst2 = jnp.concatenate([src.reshape(-1, _CH), dst.reshape(-1, _CH)])
    src64 = src.reshape(-1, 64)
    dst64 = dst.reshape(-1, 64)

    nmw = _r3(_padn(node_mask.astype(_f32)), _RN)
    nmm0 = _r3(_padn(node_mask_motif.astype(_f32)), _RN)
    nmm1 = _r3(_padn(node_mask_motif.astype(_f32), 1.0), _RN)
    emw = _r3(_pade(edge_mask.astype(_f32)), _RE)
    emm0 = _r3(_pade(edge_mask_motif.astype(_f32)), _RE)
    emm1 = _r3(_pade(edge_mask_motif.astype(_f32), 1.0), _RE)

    labn = x0
    labe = e0
    labfg = jnp.pad(_padn(label_fg), ((0, 0), (0, 128 - _FG)))
    labsc = _r3(_padn(label_scaffold[:, 0]), _RN)
    labbr = _r3(_pade(label_brics[:, 0]), _RE)

    a0p = jnp.pad(p['emb_a0'], ((0, 128 - _NA), (0, 0)))
    a1p = jnp.pad(p['emb_a1'], ((0, 128 - _NA), (0, 0)))
    b0p = jnp.pad(p['emb_b0'], ((0, 32 - _NB), (0, 0)))
    b1p = jnp.pad(p['emb_b1'], ((0, 32 - _NB), (0, 0)))

    wn3 = jnp.pad(p['Wn'], ((0, 0), (0, 128 - _NA))).reshape(3, _H, 128)
    bn = jnp.pad(p['bn'], (0, 128 - _NA)).reshape(1, 128)
    we3 = jnp.pad(p['We'], ((0, 0), (0, 32 - _NB))).reshape(3, _H, 32)
    be = jnp.pad(p['be'], (0, 32 - _NB)).reshape(1, 32)

    wf0 = p['W_fg_0'].reshape(3, _H, 384)
    bf0 = p['b_fg_0'].reshape(1, 384)
    wf1 = jnp.pad(p['W_fg_1'], ((0, 0), (0, 128 - _FG)))
    bf1 = jnp.pad(p['b_fg_1'], (0, 128 - _FG)).reshape(1, 128)
    ws0 = p['W_scaffold_0'].reshape(3, _H, 384)
    bs0 = p['b_scaffold_0'].reshape(1, 384)
    ws1 = jnp.pad(p['W_scaffold_1'], ((0, 0), (0, 127)))
    bs1 = jnp.pad(p['b_scaffold_1'], (0, 127)).reshape(1, 128)
    wb0 = p['W_brics_0'].reshape(3, _H, 384)
    bb0 = p['b_brics_0'].reshape(1, 384)
    wb1 = jnp.pad(p['W_brics_1'], ((0, 0), (0, 127)))
    bb1 = jnp.pad(p['b_brics_1'], (0, 127)).reshape(1, 128)

    zeros_h = jnp.zeros((_CH, _H), _f32)

    wm = [p['W_msg%d' % i] for i in range(_L)]
    hw, hm, zw, zm = _embed_nodes(x0, x1, nmw, nmm0, a0p, a1p, wm[0])
    eaw, eam, uw, um = _embed_edges(e0, e1, emw, emm0, b0p, b1p, wm[0])
    degp = _sc_scatter_add(jnp.ones((_EP, _H), _f32), dst, zeros_h)

    def run_pass(h, z, ea, u, emit_e, nmask, emask):
        xs, gcs, es = [], [], []
        for i in range(_L):
            emit = i < _L - 1
            wm_next = wm[i + 1] if emit else wm[i]
            parts = _sc_gather_scatter(z, u, src64, dst64, zeros_h)
            nouts = _node_update(h, parts, degp, p['W_self%d' % i], wm_next, emit)
            h = nouts[0]
            gc = _sc_gather(h, srcdst2)
            uouts = _edge_update(ea, gc, p['W_edge%d' % i], wm_next, emit)
            ea = uouts[0]
            if emit:
                z, u = nouts[1], uouts[1]
            xs.append(h); es.append(ea); gcs.append(gc)
        sn, swn = _node_ce(xs[0], xs[1], xs[2], wn3, bn, labn, nmask)
        eouts = _edge_ce(es, gcs, we3, be, labe, emask, emit_e)
        return xs, eouts, sn, swn

    xs_w, eo_w, sn_w, swn_w = run_pass(hw, zw, eaw, uw, False, nmw, emw)
    se_w, swe_w = eo_w[0], eo_w[1]
    xs_m, eo_m, sn_m, swn_m = run_pass(hm, zm, eam, um, True, nmm0, emm0)
    se_m, swe_m = eo_m[0], eo_m[1]
    ec1, ec2, ec3 = eo_m[2], eo_m[3], eo_m[4]

    sfg, ssc, swc = _node_comp(xs_m[0], xs_m[1], xs_m[2], wf0, bf0, wf1, bf1,
                               ws0, bs0, ws1, bs1, labfg, labsc, nmm1)
    sbr, swbr = _edge_comp(ec1, ec2, ec3, wb0, bb0, wb1, bb1, labbr, emm1)

    def _div(a, b):
        return (a[0, 0] / jnp.maximum(b[0, 0], 1.0)).astype(_f32)

    ln_w = _div(sn_w, swn_w)
    le_w = _div(se_w, swe_w)
    ln_m = _div(sn_m, swn_m)
    le_m = _div(se_m, swe_m)
    l_fg = _div(sfg, swc)
    l_sc = _div(ssc, swc)
    l_br = _div(sbr, swbr)

    sep = jnp.stack([ln_w, le_w, ln_m, le_m, l_fg, l_br, l_sc])
    loss = ln_w + le_w + ln_m + le_m + l_fg + l_br + l_sc
    return (loss, sep)


# speedup vs baseline: 1.1709x; 1.0244x over previous
import functools

import jax
import jax.numpy as jnp
from jax import lax
from jax.experimental import pallas as pl
from jax.experimental.pallas import tpu as pltpu
from jax.experimental.pallas import tpu_sc as plsc

_N = 10000
_E = 160000
_H = 128
_L = 3
_NA = 119
_NB = 22
_FG = 39

_NP = 10240
_EP = 163840
_NC = 2
_NS = 16
_NW = _NC * _NS
_CH = 128
_PW = _EP // _NW
_NCHUNK = _PW // _CH
_RPT = _NP // _NS
_SLAB = _RPT // _CH
_BC = 4

_RN = 512
_RE = 512
_GN = _NP // _RN
_GE = _EP // _RE

_f32 = jnp.float32
_i32 = jnp.int32


def _mesh():
    return plsc.VectorSubcoreMesh(core_axis_name="c", subcore_axis_name="s")


def _sc_gather(table, idx2):
    D = table.shape[1]
    dt = table.dtype
    R = idx2.shape[0]
    rpw = R // _NW
    nb = 4

    @functools.partial(
        pl.kernel,
        mesh=_mesh(),
        out_type=jax.ShapeDtypeStruct((R * _CH, D), dt),
        scratch_types=[pltpu.VMEM((rpw, _CH), _i32)]
        + [pltpu.VMEM((_CH, D), dt)] * nb
        + [pltpu.SemaphoreType.DMA, pltpu.SemaphoreType.DMA],
    )
    def k(table_hbm, idx_hbm, out_hbm, idx_all, *rest):
        rows = list(rest[:nb])
        sem_g, sem_w = rest[nb], rest[nb + 1]
        wid = lax.axis_index("s") * _NC + lax.axis_index("c")
        irow0 = wid * rpw
        base = pl.multiple_of(wid * rpw * _CH, 8)
        pltpu.sync_copy(idx_hbm.at[pl.ds(irow0, rpw)], idx_all)

        def g_start(j, b):
            pltpu.async_copy(table_hbm.at[idx_all.at[j]], rows[b], sem_g)

        def g_wait(j, b):
            pltpu.make_async_copy(table_hbm.at[idx_all.at[j]], rows[b], sem_g).wait()

        def w_start(j, b):
            off = pl.multiple_of(base + j * _CH, 8)
            pltpu.async_copy(rows[b], out_hbm.at[pl.ds(off, _CH)], sem_w)

        def w_wait(j, b):
            off = pl.multiple_of(base + j * _CH, 8)
            pltpu.make_async_copy(rows[b], out_hbm.at[pl.ds(off, _CH)], sem_w).wait()

        for b in range(nb):
            g_start(b, b)

        def body(t, carry):
            j0 = t * nb
            for b in range(nb):
                g_wait(j0 + b, b)
                w_start(j0 + b, b)
            for b in range(nb):
                w_wait(j0 + b, b)
                g_start(j0 + nb + b, b)
            return carry

        lax.fori_loop(0, rpw // nb - 1, body, 0)
        jl = rpw - nb
        for b in range(nb):
            g_wait(jl + b, b)
            w_start(jl + b, b)
        for b in range(nb):
            w_wait(jl + b, b)

    return k(table, idx2)


def _sc_scatter_add(vals, idx, zeros_h):
    D = vals.shape[1]

    @functools.partial(
        pl.kernel,
        mesh=_mesh(),
        out_type=jax.ShapeDtypeStruct((2 * _NP, D), _f32),
        scratch_types=[
            pltpu.VMEM((_CH,), _i32),
            pltpu.VMEM((_CH,), _i32),
            pltpu.VMEM((_CH, D), _f32),
            pltpu.VMEM((_CH, D), _f32),
            pltpu.VMEM_SHARED((_NP, D), _f32),
            pltpu.SemaphoreType.DMA,
            pltpu.SemaphoreType.DMA,
            pltpu.SemaphoreType.DMA,
        ],
    )
    def k(vals_hbm, idx_hbm, zeros_hbm, out_hbm,
          i0, i1, v0, v1, acc_sh, sem_i, sem_l, sem_s):
        cid = lax.axis_index("c")
        sid = lax.axis_index("s")
        wid = sid * _NC + cid
        r0 = sid * _RPT
        idxs = [i0, i1]
        bufs = [v0, v1]
        nb = 2
        pltpu.sync_copy(zeros_hbm, v0)

        def zbody(kk, carry):
            pltpu.sync_copy(v0, acc_sh.at[pl.ds(pl.multiple_of(r0 + kk * _CH, 8), _CH)])
            return carry

        lax.fori_loop(0, _SLAB, zbody, 0)
        plsc.subcore_barrier()

        base = pl.multiple_of(wid * _PW, 8)

        def l_start(j, b):
            off = pl.multiple_of(base + j * _CH, 8)
            pltpu.async_copy(idx_hbm.at[pl.ds(off, _CH)], idxs[b], sem_i)
            pltpu.async_copy(vals_hbm.at[pl.ds(off, _CH)], bufs[b], sem_l)

        def l_wait(j, b):
            off = pl.multiple_of(base + j * _CH, 8)
            pltpu.make_async_copy(idx_hbm.at[pl.ds(off, _CH)], idxs[b], sem_i).wait()
            pltpu.make_async_copy(vals_hbm.at[pl.ds(off, _CH)], bufs[b], sem_l).wait()

        def s_start(b):
            pltpu.async_copy(bufs[b], acc_sh.at[idxs[b]], sem_s, add=True)

        def s_wait(b):
            pltpu.make_async_copy(bufs[b], acc_sh.at[idxs[b]], sem_s).wait()

        for b in range(nb):
            l_start(b, b)

        def body(t, carry):
            j0 = t * nb
            for b in range(nb):
                l_wait(j0 + b, b)
                s_start(b)
            for b in range(nb):
                s_wait(b)
                l_start(j0 + nb + b, b)
            return carry

        lax.fori_loop(0, _NCHUNK // nb - 1, body, 0)
        jl = _NCHUNK - nb
        for b in range(nb):
            l_wait(jl + b, b)
            s_start(b)
        for b in range(nb):
            s_wait(b)
        plsc.subcore_barrier()

        def obody(kk, carry):
            src_off = pl.multiple_of(r0 + kk * _CH, 8)
            dst_off = pl.multiple_of(cid * _NP + r0 + kk * _CH, 8)
            pltpu.sync_copy(acc_sh.at[pl.ds(src_off, _CH)], v0)
            pltpu.sync_copy(v0, out_hbm.at[pl.ds(dst_off, _CH)])
            return carry

        lax.fori_loop(0, _SLAB, obody, 0)

    return k(vals, idx, zeros_h)


def _sc_degree(dst64, zeros_h, ones_h):
    ch = 64
    nchunk = _PW // ch

    @functools.partial(
        pl.kernel,
        mesh=_mesh(),
        out_type=jax.ShapeDtypeStruct((2 * _NP, _H), _f32),
        scratch_types=[
            pltpu.VMEM((nchunk, ch), _i32),
            pltpu.VMEM((ch, _H), _f32),
            pltpu.VMEM_SHARED((_NP, _H), _f32),
        ],
    )
    def k(dst_hbm, zeros_hbm, ones_hbm, out_hbm, didx, buf, acc_sh):
        cid = lax.axis_index("c")
        sid = lax.axis_index("s")
        wid = sid * _NC + cid
        r0 = sid * _RPT
        irow0 = wid * nchunk
        pltpu.sync_copy(dst_hbm.at[pl.ds(irow0, nchunk)], didx)
        pltpu.sync_copy(zeros_hbm.at[pl.ds(0, ch)], buf)

        def zbody(kk, carry):
            pltpu.sync_copy(buf, acc_sh.at[pl.ds(pl.multiple_of(r0 + kk * ch, 8), ch)])
            return carry

        lax.fori_loop(0, _RPT // ch, zbody, 0)
        plsc.subcore_barrier()
        pltpu.sync_copy(ones_hbm, buf)

        def sbody(j, carry):
            pltpu.sync_copy(buf, acc_sh.at[didx.at[j]], add=True)
            return carry

        lax.fori_loop(0, nchunk, sbody, 0)
        plsc.subcore_barrier()

        def obody(kk, carry):
            src_off = pl.multiple_of(r0 + kk * ch, 8)
            dst_off = pl.multiple_of(cid * _NP + r0 + kk * ch, 8)
            pltpu.sync_copy(acc_sh.at[pl.ds(src_off, ch)], buf)
            pltpu.sync_copy(buf, out_hbm.at[pl.ds(dst_off, ch)])
            return carry

        lax.fori_loop(0, _RPT // ch, obody, 0)

    return k(dst64, zeros_h, ones_h)


def _sc_gather_scatter(z, u, src64, dst64, zeros_h):
    D = z.shape[1]
    ch = 64
    nchunk = _PW // ch

    @functools.partial(
        pl.kernel,
        mesh=_mesh(),
        out_type=jax.ShapeDtypeStruct((2 * _NP, D), _f32),
        scratch_types=[
            pltpu.VMEM((nchunk // 2, ch), _i32),
            pltpu.VMEM((nchunk // 2, ch), _i32),
            pltpu.VMEM((ch, D), _f32),
            pltpu.VMEM((ch, D), _f32),
            pltpu.VMEM((ch, D), _f32),
            pltpu.VMEM((ch, D), _f32),
            pltpu.VMEM_SHARED((_NP, D), _f32),
            pltpu.SemaphoreType.DMA,
            pltpu.SemaphoreType.DMA,
            pltpu.SemaphoreType.DMA,
        ],
    )
    def k(z_hbm, u_hbm, src_hbm, dst_hbm, zeros_hbm, out_hbm,
          sidx, didx, g0, g1, u0, u1, acc_sh, sem_g, sem_l, sem_s):
        cid = lax.axis_index("c")
        sid = lax.axis_index("s")
        wid = sid * _NC + cid
        r0 = sid * _RPT
        gbufs = [g0, g1]
        ubufs = [u0, u1]
        nb = 2
        irow0 = wid * nchunk
        pltpu.sync_copy(zeros_hbm.at[pl.ds(0, ch)], g0)

        def zbody(kk, carry):
            pltpu.sync_copy(g0, acc_sh.at[pl.ds(pl.multiple_of(r0 + kk * ch, 8), ch)])
            return carry

        lax.fori_loop(0, _RPT // ch, zbody, 0)
        plsc.subcore_barrier()

        base = pl.multiple_of(wid * _PW, 8)
        nh = nchunk // 2

        for ph in range(2):
            jb = ph * nh
            pltpu.sync_copy(src_hbm.at[pl.ds(irow0 + jb, nh)], sidx)
            pltpu.sync_copy(dst_hbm.at[pl.ds(irow0 + jb, nh)], didx)

            def l_start(j, b, jb=jb):
                off = pl.multiple_of(base + (jb + j) * ch, 8)
                pltpu.async_copy(z_hbm.at[sidx.at[j]], gbufs[b], sem_g)
                pltpu.async_copy(u_hbm.at[pl.ds(off, ch)], ubufs[b], sem_l)

            def l_wait(j, b, jb=jb):
                off = pl.multiple_of(base + (jb + j) * ch, 8)
                pltpu.make_async_copy(z_hbm.at[sidx.at[j]], gbufs[b], sem_g).wait()
                pltpu.make_async_copy(u_hbm.at[pl.ds(off, ch)], ubufs[b], sem_l).wait()

            def s_start(j, b):
                pltpu.async_copy(gbufs[b], acc_sh.at[didx.at[j]], sem_s, add=True)
                pltpu.async_copy(ubufs[b], acc_sh.at[didx.at[j]], sem_s, add=True)

            def s_wait(j, b):
                pltpu.make_async_copy(gbufs[b], acc_sh.at[didx.at[j]], sem_s).wait()
                pltpu.make_async_copy(ubufs[b], acc_sh.at[didx.at[j]], sem_s).wait()

            for b in range(nb):
                l_start(b, b)

            def body(t, carry, l_start=l_start, l_wait=l_wait,
                     s_start=s_start, s_wait=s_wait):
                j0 = t * nb
                for b in range(nb):
                    l_wait(j0 + b, b)
                    s_start(j0 + b, b)
                for b in range(nb):
                    s_wait(j0 + b, b)
                    l_start(j0 + nb + b, b)
                return carry

            lax.fori_loop(0, nh // nb - 1, body, 0)
            jl = nh - nb
            for b in range(nb):
                l_wait(jl + b, b)
                s_start(jl + b, b)
            for b in range(nb):
                s_wait(jl + b, b)
        plsc.subcore_barrier()

        def obody(kk, carry):
            src_off = pl.multiple_of(r0 + kk * ch, 8)
            dst_off = pl.multiple_of(cid * _NP + r0 + kk * ch, 8)
            pltpu.sync_copy(acc_sh.at[pl.ds(src_off, ch)], g0)
            pltpu.sync_copy(g0, out_hbm.at[pl.ds(dst_off, ch)])
            return carry

        lax.fori_loop(0, _RPT // ch, obody, 0)

    return k(z, u, src64, dst64, zeros_h)


def _cparams():
    return pltpu.CompilerParams(dimension_semantics=("arbitrary",))


def _onehot_dot(idx, table_ref, ncls):
    oh = (idx[:, None] == lax.broadcasted_iota(_i32, (idx.shape[0], ncls), 1)).astype(_f32)
    return jax.lax.dot_general(oh, table_ref[...], (((1,), (0,)), ((), ())),
                               precision=jax.lax.Precision.HIGHEST,
                               preferred_element_type=_f32)


def _embed_nodes(x0, x1, nmw, nmm, a0p, a1p, wm0):
    def body(x0_r, x1_r, mw_r, mm_r, a0_r, a1_r, wm_r, ow_r, om_r, ozw_r, ozm_r):
        i0 = x0_r[0, 0, :]
        i1 = x1_r[0, 0, :]
        h = _onehot_dot(i0, a0_r, 128) + _onehot_dot(i1, a1_r, 128)
        hw = h * (1.0 - mw_r[0, 0, :])[:, None]
        hm = h * (1.0 - mm_r[0, 0, :])[:, None]
        ow_r[...] = hw
        om_r[...] = hm
        ozw_r[...] = jnp.dot(hw, wm_r[...], preferred_element_type=_f32)
        ozm_r[...] = jnp.dot(hm, wm_r[...], preferred_element_type=_f32)

    sh = jax.ShapeDtypeStruct((_NP, _H), _f32)
    v3 = pl.BlockSpec((1, 1, _RN), lambda i: (i, 0, 0))
    full = pl.BlockSpec((128, _H), lambda i: (0, 0))
    blkh = pl.BlockSpec((_RN, _H), lambda i: (i, 0))
    return pl.pallas_call(
        body, grid=(_GN,),
        in_specs=[v3, v3, v3, v3, full, full, full],
        out_specs=[blkh] * 4,
        out_shape=[sh] * 4, compiler_params=_cparams(),
    )(x0, x1, nmw, nmm, a0p, a1p, wm0)


def _embed_edges(e0, e1, emw, emm, b0p, b1p, wm0):
    def body(e0_r, e1_r, mw_r, mm_r, b0_r, b1_r, wm_r, ow_r, om_r, ouw_r, oum_r):
        i0 = e0_r[0, 0, :]
        i1 = e1_r[0, 0, :]
        h = _onehot_dot(i0, b0_r, 32) + _onehot_dot(i1, b1_r, 32)
        ew = h * (1.0 - mw_r[0, 0, :])[:, None]
        em = h * (1.0 - mm_r[0, 0, :])[:, None]
        ow_r[...] = ew
        om_r[...] = em
        ouw_r[...] = jnp.dot(ew, wm_r[...], preferred_element_type=_f32)
        oum_r[...] = jnp.dot(em, wm_r[...], preferred_element_type=_f32)

    sh = jax.ShapeDtypeStruct((_EP, _H), _f32)
    v3 = pl.BlockSpec((1, 1, _RE), lambda i: (i, 0, 0))
    full = pl.BlockSpec((32, _H), lambda i: (0, 0))
    wfull = pl.BlockSpec((_H, _H), lambda i: (0, 0))
    blk = pl.BlockSpec((_RE, _H), lambda i: (i, 0))
    return pl.pallas_call(
        body, grid=(_GE,),
        in_specs=[v3, v3, v3, v3, full, full, wfull],
        out_specs=[blk] * 4,
        out_shape=[sh] * 4, compiler_params=_cparams(),
    )(e0, e1, emw, emm, b0p, b1p, wm0)


def _node_update(h, parts, degp, w, wm_next, emit_z):
    def body(h_r, p0_r, p1_r, d0_r, d1_r, w_r, wm_r, o_r, *outs):
        agg = p0_r[...] + p1_r[...]
        deg = d0_r[:, 0:1] + d1_r[:, 0:1]
        inv = 1.0 / jnp.maximum(deg, 1.0)
        hn = jnp.maximum(
            jnp.dot(h_r[...], w_r[...], preferred_element_type=_f32) + agg * inv, 0.0)
        o_r[...] = hn
        if emit_z:
            outs[0][...] = jnp.dot(hn, wm_r[...], preferred_element_type=_f32)

    blk = pl.BlockSpec((_RN, _H), lambda i: (i, 0))
    p0 = pl.BlockSpec((_RN, _H), lambda i: (i, 0))
    p1 = pl.BlockSpec((_RN, _H), lambda i: (i + _GN, 0))
    d0 = pl.BlockSpec((_RN, _H), lambda i: (i, 0))
    d1 = pl.BlockSpec((_RN, _H), lambda i: (i + _GN, 0))
    full = pl.BlockSpec((_H, _H), lambda i: (0, 0))
    sh = jax.ShapeDtypeStruct((_NP, _H), _f32)
    out_specs = [blk] + ([blk] if emit_z else [])
    out_shape = [sh] + ([sh] if emit_z else [])
    return pl.pallas_call(
        body, grid=(_GN,), in_specs=[blk, p0, p1, d0, d1, full, full],
        out_specs=out_specs, out_shape=out_shape,
        compiler_params=_cparams(),
    )(h, parts, parts, degp, degp, w, wm_next)


def _edge_update(ea, gc, w, wm_next, emit_u):
    def body(ea_r, ga_r, gb_r, w_r, wm_r, o_r, *outs):
        en = jnp.maximum(
            jnp.dot(ea_r[...], w_r[...], preferred_element_type=_f32)
            + ga_r[...] + gb_r[...], 0.0)
        o_r[...] = en
        if emit_u:
            outs[0][...] = jnp.dot(en, wm_r[...], preferred_element_type=_f32)

    blk = pl.BlockSpec((_RE, _H), lambda i: (i, 0))
    blka = pl.BlockSpec((_RE, _H), lambda i: (i, 0))
    blkb = pl.BlockSpec((_RE, _H), lambda i: (i + _GE, 0))
    full = pl.BlockSpec((_H, _H), lambda i: (0, 0))
    sh = jax.ShapeDtypeStruct((_EP, _H), _f32)
    out_specs = [blk] + ([blk] if emit_u else [])
    out_shape = [sh] + ([sh] if emit_u else [])
    return pl.pallas_call(
        body, grid=(_GE,), in_specs=[blk, blka, blkb, full, full],
        out_specs=out_specs, out_shape=out_shape, compiler_params=_cparams(),
    )(ea, gc, gc, w, wm_next)


def _acc2(ref, val, i):
    @pl.when(i == 0)
    def _():
        ref[...] = jnp.zeros((1, 1), _f32)

    ref[...] += jnp.reshape(val, (1, 1))


def _ce_from_logits(z, lab, ncls_real):
    cm = lax.broadcasted_iota(_i32, z.shape, 1) < ncls_real
    z = jnp.where(cm, z, -1e30)
    m = jnp.max(z, axis=1, keepdims=True)
    lse = m + jnp.log(jnp.sum(jnp.exp(z - m), axis=1, keepdims=True))
    oh = lab[:, None] == lax.broadcasted_iota(_i32, z.shape, 1)
    zy = jnp.sum(jnp.where(oh, z, 0.0), axis=1, keepdims=True)
    return lse - zy


def _node_ce(x1, x2, x3, wn3, bn, lab3, w3):
    def body(x1_r, x2_r, x3_r, w_r, b_r, lab_r, wt_r, s_r, sw_r):
        i = pl.program_id(0)
        z = (jnp.dot(x1_r[...], w_r[0], preferred_element_type=_f32)
             + jnp.dot(x2_r[...], w_r[1], preferred_element_type=_f32)
             + jnp.dot(x3_r[...], w_r[2], preferred_element_type=_f32)
             + b_r[...])
        nll = _ce_from_logits(z, lab_r[0, 0, :], _NA)
        wt = wt_r[0, 0, :][:, None]
        _acc2(s_r, jnp.sum(nll * wt), i)
        _acc2(sw_r, jnp.sum(wt), i)

    blk = pl.BlockSpec((_RN, _H), lambda i: (i, 0))
    wfull = pl.BlockSpec((3, _H, 128), lambda i: (0, 0, 0))
    bfull = pl.BlockSpec((1, 128), lambda i: (0, 0))
    v3 = pl.BlockSpec((1, 1, _RN), lambda i: (i, 0, 0))
    sc = pl.BlockSpec((1, 1), lambda i: (0, 0))
    s1 = jax.ShapeDtypeStruct((1, 1), _f32)
    return pl.pallas_call(
        body, grid=(_GN,), in_specs=[blk, blk, blk, wfull, bfull, v3, v3],
        out_specs=[sc, sc], out_shape=[s1, s1], compiler_params=_cparams(),
    )(x1, x2, x3, wn3, bn, lab3, w3)


def _edge_ce(es, gcs, we3, be, lab3, w3, emit_e):
    def body(e1_r, e2_r, e3_r, a1_r, a2_r, a3_r, b1_r, b2_r, b3_r,
             w_r, b_r, lab_r, wt_r, *outs):
        i = pl.program_id(0)
        c1 = e1_r[...] + a1_r[...] + b1_r[...]
        c2 = e2_r[...] + a2_r[...] + b2_r[...]
        c3 = e3_r[...] + a3_r[...] + b3_r[...]
        z = (jnp.dot(c1, w_r[0], preferred_element_type=_f32)
             + jnp.dot(c2, w_r[1], preferred_element_type=_f32)
             + jnp.dot(c3, w_r[2], preferred_element_type=_f32)
             + b_r[...])
        nll = _ce_from_logits(z, lab_r[0, 0, :], _NB)
        wt = wt_r[0, 0, :][:, None]
        _acc2(outs[0], jnp.sum(nll * wt), i)
        _acc2(outs[1], jnp.sum(wt), i)
        if emit_e:
            outs[2][...] = c1
            outs[3][...] = c2
            outs[4][...] = c3

    blk = pl.BlockSpec((_RE, _H), lambda i: (i, 0))
    blkb = pl.BlockSpec((_RE, _H), lambda i: (i + _GE, 0))
    wfull = pl.BlockSpec((3, _H, 32), lambda i: (0, 0, 0))
    bfull = pl.BlockSpec((1, 32), lambda i: (0, 0))
    v3 = pl.BlockSpec((1, 1, _RE), lambda i: (i, 0, 0))
    sc = pl.BlockSpec((1, 1), lambda i: (0, 0))
    s1 = jax.ShapeDtypeStruct((1, 1), _f32)
    eo = jax.ShapeDtypeStruct((_EP, _H), _f32)
    out_specs = [sc, sc] + ([blk] * 3 if emit_e else [])
    out_shape = [s1, s1] + ([eo] * 3 if emit_e else [])
    return pl.pallas_call(
        body, grid=(_GE,),
        in_specs=[blk] * 3 + [blk] * 3 + [blkb] * 3 + [wfull, bfull, v3, v3],
        out_specs=out_specs, out_shape=out_shape, compiler_params=_cparams(),
    )(es[0], es[1], es[2], gcs[0], gcs[1], gcs[2], gcs[0], gcs[1], gcs[2],
      we3, be, lab3, w3)


def _bce_elem(z, y):
    return jnp.maximum(z, 0.0) - z * y + jnp.log(1.0 + jnp.exp(-jnp.abs(z)))


def _node_comp(x1, x2, x3, wf0, bf0, wf1, bf1, ws0, bs0, ws1, bs1,
               labfg, labsc3, nmm13):
    def body(x1_r, x2_r, x3_r, wf0_r, bf0_r, wf1_r, bf1_r,
             ws0_r, bs0_r, ws1_r, bs1_r, yfg_r, ysc_r, m_r,
             sfg_r, ssc_r, sw_r):
        i = pl.program_id(0)
        w = (1.0 - m_r[0, 0, :])[:, None]

        hf = (jnp.dot(x1_r[...], wf0_r[0], preferred_element_type=_f32)
              + jnp.dot(x2_r[...], wf0_r[1], preferred_element_type=_f32)
              + jnp.dot(x3_r[...], wf0_r[2], preferred_element_type=_f32)
              + bf0_r[...])
        hf = jnp.maximum(hf, 0.0)
        zf = jnp.dot(hf, wf1_r[...], preferred_element_type=_f32) + bf1_r[...]
        cmf = (lax.broadcasted_iota(_i32, zf.shape, 1) < _FG).astype(_f32)
        perf = jnp.sum(_bce_elem(zf, yfg_r[...]) * cmf, axis=1, keepdims=True) / _FG

        hs = (jnp.dot(x1_r[...], ws0_r[0], preferred_element_type=_f32)
              + jnp.dot(x2_r[...], ws0_r[1], preferred_element_type=_f32)
              + jnp.dot(x3_r[...], ws0_r[2], preferred_element_type=_f32)
              + bs0_r[...])
        hs = jnp.maximum(hs, 0.0)
        zs = jnp.dot(hs, ws1_r[...], preferred_element_type=_f32) + bs1_r[...]
        ysc = ysc_r[0, 0, :][:, None]
        pers = _bce_elem(zs[:, 0:1], ysc)
        _acc2(sfg_r, jnp.sum(perf * w), i)
        _acc2(ssc_r, jnp.sum(pers * w), i)
        _acc2(sw_r, jnp.sum(w), i)

    blk = pl.BlockSpec((_RN, _H), lambda i: (i, 0))
    w0full = pl.BlockSpec((3, _H, 384), lambda i: (0, 0, 0))
    b0full = pl.BlockSpec((1, 384), lambda i: (0, 0))
    w1full = pl.BlockSpec((384, 128), lambda i: (0, 0))
    b1full = pl.BlockSpec((1, 128), lambda i: (0, 0))
    yfull = pl.BlockSpec((_RN, 128), lambda i: (i, 0))
    v3 = pl.BlockSpec((1, 1, _RN), lambda i: (i, 0, 0))
    sc = pl.BlockSpec((1, 1), lambda i: (0, 0))
    s1 = jax.ShapeDtypeStruct((1, 1), _f32)
    return pl.pallas_call(
        body, grid=(_GN,),
        in_specs=[blk, blk, blk, w0full, b0full, w1full, b1full,
                  w0full, b0full, w1full, b1full, yfull, v3, v3],
        out_specs=[sc, sc, sc], out_shape=[s1, s1, s1], compiler_params=_cparams(),
    )(x1, x2, x3, wf0, bf0, wf1, bf1, ws0, bs0, ws1, bs1, labfg, labsc3, nmm13)


def _edge_comp(e1, e2, e3, wb0, bb0, wb1, bb1, labbr3, emm13):
    def body(e1_r, e2_r, e3_r, w0_r, b0_r, w1_r, b1_r, y_r, m_r, s_r, sw_r):
        i = pl.program_id(0)
        w = (1.0 - m_r[0, 0, :])[:, None]
        hh = (jnp.dot(e1_r[...], w0_r[0], preferred_element_type=_f32)
              + jnp.dot(e2_r[...], w0_r[1], preferred_element_type=_f32)
              + jnp.dot(e3_r[...], w0_r[2], preferred_element_type=_f32)
              + b0_r[...])
        hh = jnp.maximum(hh, 0.0)
        z = jnp.dot(hh, w1_r[...], preferred_element_type=_f32) + b1_r[...]
        y = y_r[0, 0, :][:, None]
        per = _bce_elem(z[:, 0:1], y)
        _acc2(s_r, jnp.sum(per * w), i)
        _acc2(sw_r, jnp.sum(w), i)

    blk = pl.BlockSpec((_RE, _H), lambda i: (i, 0))
    w0full = pl.BlockSpec((3, _H, 384), lambda i: (0, 0, 0))
    b0full = pl.BlockSpec((1, 384), lambda i: (0, 0))
    w1full = pl.BlockSpec((384, 128), lambda i: (0, 0))
    b1full = pl.BlockSpec((1, 128), lambda i: (0, 0))
    v3 = pl.BlockSpec((1, 1, _RE), lambda i: (i, 0, 0))
    sc = pl.BlockSpec((1, 1), lambda i: (0, 0))
    s1 = jax.ShapeDtypeStruct((1, 1), _f32)
    return pl.pallas_call(
        body, grid=(_GE,),
        in_specs=[blk, blk, blk, w0full, b0full, w1full, b1full, v3, v3],
        out_specs=[sc, sc], out_shape=[s1, s1], compiler_params=_cparams(),
    )(e1, e2, e3, wb0, bb0, wb1, bb1, labbr3, emm13)


def _padn(v, fill=0):
    return jnp.pad(v, ((0, _NP - _N),) + ((0, 0),) * (v.ndim - 1), constant_values=fill)


def _pade(v, fill=0):
    return jnp.pad(v, ((0, _EP - _E),) + ((0, 0),) * (v.ndim - 1), constant_values=fill)


def _r3(v, r):
    return v.reshape(-1, 1, r)


def kernel(x, edge_attr, edge_index, node_mask, edge_mask, node_mask_motif,
           edge_mask_motif, label_fg, label_brics, label_scaffold, params):
    p = params

    x0 = _r3(_padn(x[:, 0].astype(_i32)), _RN)
    x1 = _r3(_padn(x[:, 1].astype(_i32)), _RN)
    e0 = _r3(_pade(edge_attr[:, 0].astype(_i32)), _RE)
    e1 = _r3(_pade(edge_attr[:, 1].astype(_i32)), _RE)
    src = _pade(edge_index[0].astype(_i32), 0)
    dst = _pade(edge_index[1].astype(_i32), _NP - 1)
    srcdst2 = jnp.concatenate([src.reshape(-1, _CH), dst.reshape(-1, _CH)])
    src64 = src.reshape(-1, 64)
    dst64 = dst.reshape(-1, 64)

    nmw = _r3(_padn(node_mask.astype(_f32)), _RN)
    nmm0 = _r3(_padn(node_mask_motif.astype(_f32)), _RN)
    nmm1 = _r3(_padn(node_mask_motif.astype(_f32), 1.0), _RN)
    emw = _r3(_pade(edge_mask.astype(_f32)), _RE)
    emm0 = _r3(_pade(edge_mask_motif.astype(_f32)), _RE)
    emm1 = _r3(_pade(edge_mask_motif.astype(_f32), 1.0), _RE)

    labn = x0
    labe = e0
    labfg = jnp.pad(_padn(label_fg), ((0, 0), (0, 128 - _FG)))
    labsc = _r3(_padn(label_scaffold[:, 0]), _RN)
    labbr = _r3(_pade(label_brics[:, 0]), _RE)

    a0p = jnp.pad(p['emb_a0'], ((0, 128 - _NA), (0, 0)))
    a1p = jnp.pad(p['emb_a1'], ((0, 128 - _NA), (0, 0)))
    b0p = jnp.pad(p['emb_b0'], ((0, 32 - _NB), (0, 0)))
    b1p = jnp.pad(p['emb_b1'], ((0, 32 - _NB), (0, 0)))

    wn3 = jnp.pad(p['Wn'], ((0, 0), (0, 128 - _NA))).reshape(3, _H, 128)
    bn = jnp.pad(p['bn'], (0, 128 - _NA)).reshape(1, 128)
    we3 = jnp.pad(p['We'], ((0, 0), (0, 32 - _NB))).reshape(3, _H, 32)
    be = jnp.pad(p['be'], (0, 32 - _NB)).reshape(1, 32)

    wf0 = p['W_fg_0'].reshape(3, _H, 384)
    bf0 = p['b_fg_0'].reshape(1, 384)
    wf1 = jnp.pad(p['W_fg_1'], ((0, 0), (0, 128 - _FG)))
    bf1 = jnp.pad(p['b_fg_1'], (0, 128 - _FG)).reshape(1, 128)
    ws0 = p['W_scaffold_0'].reshape(3, _H, 384)
    bs0 = p['b_scaffold_0'].reshape(1, 384)
    ws1 = jnp.pad(p['W_scaffold_1'], ((0, 0), (0, 127)))
    bs1 = jnp.pad(p['b_scaffold_1'], (0, 127)).reshape(1, 128)
    wb0 = p['W_brics_0'].reshape(3, _H, 384)
    bb0 = p['b_brics_0'].reshape(1, 384)
    wb1 = jnp.pad(p['W_brics_1'], ((0, 0), (0, 127)))
    bb1 = jnp.pad(p['b_brics_1'], (0, 127)).reshape(1, 128)

    zeros_h = jnp.zeros((_CH, _H), _f32)

    wm = [p['W_msg%d' % i] for i in range(_L)]
    hw, hm, zw, zm = _embed_nodes(x0, x1, nmw, nmm0, a0p, a1p, wm[0])
    eaw, eam, uw, um = _embed_edges(e0, e1, emw, emm0, b0p, b1p, wm[0])
    degp = _sc_degree(dst64, zeros_h, jnp.ones((64, _H), _f32))

    def run_pass(h, z, ea, u, emit_e, nmask, emask):
        xs, gcs, es = [], [], []
        for i in range(_L):
            emit = i < _L - 1
            wm_next = wm[i + 1] if emit else wm[i]
            parts = _sc_gather_scatter(z, u, src64, dst64, zeros_h)
            nouts = _node_update(h, parts, degp, p['W_self%d' % i], wm_next, emit)
            h = nouts[0]
            gc = _sc_gather(h, srcdst2)
            uouts = _edge_update(ea, gc, p['W_edge%d' % i], wm_next, emit)
            ea = uouts[0]
            if emit:
                z, u = nouts[1], uouts[1]
            xs.append(h); es.append(ea); gcs.append(gc)
        sn, swn = _node_ce(xs[0], xs[1], xs[2], wn3, bn, labn, nmask)
        eouts = _edge_ce(es, gcs, we3, be, labe, emask, emit_e)
        return xs, eouts, sn, swn

    xs_w, eo_w, sn_w, swn_w = run_pass(hw, zw, eaw, uw, False, nmw, emw)
    se_w, swe_w = eo_w[0], eo_w[1]
    xs_m, eo_m, sn_m, swn_m = run_pass(hm, zm, eam, um, True, nmm0, emm0)
    se_m, swe_m = eo_m[0], eo_m[1]
    ec1, ec2, ec3 = eo_m[2], eo_m[3], eo_m[4]

    sfg, ssc, swc = _node_comp(xs_m[0], xs_m[1], xs_m[2], wf0, bf0, wf1, bf1,
                               ws0, bs0, ws1, bs1, labfg, labsc, nmm1)
    sbr, swbr = _edge_comp(ec1, ec2, ec3, wb0, bb0, wb1, bb1, labbr, emm1)

    def _div(a, b):
        return (a[0, 0] / jnp.maximum(b[0, 0], 1.0)).astype(_f32)

    ln_w = _div(sn_w, swn_w)
    le_w = _div(se_w, swe_w)
    ln_m = _div(sn_m, swn_m)
    le_m = _div(se_m, swe_m)
    l_fg = _div(sfg, swc)
    l_sc = _div(ssc, swc)
    l_br = _div(sbr, swbr)

    sep = jnp.stack([ln_w, le_w, ln_m, le_m, l_fg, l_br, l_sc])
    loss = ln_w + le_w + ln_m + le_m + l_fg + l_br + l_sc
    return (loss, sep)


# speedup vs baseline: 1.1711x; 1.0002x over previous
import functools

import jax
import jax.numpy as jnp
from jax import lax
from jax.experimental import pallas as pl
from jax.experimental.pallas import tpu as pltpu
from jax.experimental.pallas import tpu_sc as plsc

_N = 10000
_E = 160000
_H = 128
_L = 3
_NA = 119
_NB = 22
_FG = 39

_NP = 10240
_EP = 163840
_NC = 2
_NS = 16
_NW = _NC * _NS
_CH = 128
_PW = _EP // _NW
_NCHUNK = _PW // _CH
_RPT = _NP // _NS
_SLAB = _RPT // _CH
_BC = 4

_RN = 512
_RE = 512
_GN = _NP // _RN
_GE = _EP // _RE

_f32 = jnp.float32
_i32 = jnp.int32


def _mesh():
    return plsc.VectorSubcoreMesh(core_axis_name="c", subcore_axis_name="s")


def _sc_gather(table, idx2):
    D = table.shape[1]
    dt = table.dtype
    R = idx2.shape[0]
    rpw = R // _NW
    nb = 5

    @functools.partial(
        pl.kernel,
        mesh=_mesh(),
        out_type=jax.ShapeDtypeStruct((R * _CH, D), dt),
        scratch_types=[pltpu.VMEM((rpw, _CH), _i32)]
        + [pltpu.VMEM((_CH, D), dt)] * nb
        + [pltpu.SemaphoreType.DMA, pltpu.SemaphoreType.DMA],
    )
    def k(table_hbm, idx_hbm, out_hbm, idx_all, *rest):
        rows = list(rest[:nb])
        sem_g, sem_w = rest[nb], rest[nb + 1]
        wid = lax.axis_index("s") * _NC + lax.axis_index("c")
        irow0 = wid * rpw
        base = pl.multiple_of(wid * rpw * _CH, 8)
        pltpu.sync_copy(idx_hbm.at[pl.ds(irow0, rpw)], idx_all)

        def g_start(j, b):
            pltpu.async_copy(table_hbm.at[idx_all.at[j]], rows[b], sem_g)

        def g_wait(j, b):
            pltpu.make_async_copy(table_hbm.at[idx_all.at[j]], rows[b], sem_g).wait()

        def w_start(j, b):
            off = pl.multiple_of(base + j * _CH, 8)
            pltpu.async_copy(rows[b], out_hbm.at[pl.ds(off, _CH)], sem_w)

        def w_wait(j, b):
            off = pl.multiple_of(base + j * _CH, 8)
            pltpu.make_async_copy(rows[b], out_hbm.at[pl.ds(off, _CH)], sem_w).wait()

        for b in range(nb):
            g_start(b, b)

        def body(t, carry):
            j0 = t * nb
            for b in range(nb):
                g_wait(j0 + b, b)
                w_start(j0 + b, b)
            for b in range(nb):
                w_wait(j0 + b, b)
                g_start(j0 + nb + b, b)
            return carry

        lax.fori_loop(0, rpw // nb - 1, body, 0)
        jl = rpw - nb
        for b in range(nb):
            g_wait(jl + b, b)
            w_start(jl + b, b)
        for b in range(nb):
            w_wait(jl + b, b)

    return k(table, idx2)


def _sc_scatter_add(vals, idx, zeros_h):
    D = vals.shape[1]

    @functools.partial(
        pl.kernel,
        mesh=_mesh(),
        out_type=jax.ShapeDtypeStruct((2 * _NP, D), _f32),
        scratch_types=[
            pltpu.VMEM((_CH,), _i32),
            pltpu.VMEM((_CH,), _i32),
            pltpu.VMEM((_CH, D), _f32),
            pltpu.VMEM((_CH, D), _f32),
            pltpu.VMEM_SHARED((_NP, D), _f32),
            pltpu.SemaphoreType.DMA,
            pltpu.SemaphoreType.DMA,
            pltpu.SemaphoreType.DMA,
        ],
    )
    def k(vals_hbm, idx_hbm, zeros_hbm, out_hbm,
          i0, i1, v0, v1, acc_sh, sem_i, sem_l, sem_s):
        cid = lax.axis_index("c")
        sid = lax.axis_index("s")
        wid = sid * _NC + cid
        r0 = sid * _RPT
        idxs = [i0, i1]
        bufs = [v0, v1]
        nb = 2
        pltpu.sync_copy(zeros_hbm, v0)

        def zbody(kk, carry):
            pltpu.sync_copy(v0, acc_sh.at[pl.ds(pl.multiple_of(r0 + kk * _CH, 8), _CH)])
            return carry

        lax.fori_loop(0, _SLAB, zbody, 0)
        plsc.subcore_barrier()

        base = pl.multiple_of(wid * _PW, 8)

        def l_start(j, b):
            off = pl.multiple_of(base + j * _CH, 8)
            pltpu.async_copy(idx_hbm.at[pl.ds(off, _CH)], idxs[b], sem_i)
            pltpu.async_copy(vals_hbm.at[pl.ds(off, _CH)], bufs[b], sem_l)

        def l_wait(j, b):
            off = pl.multiple_of(base + j * _CH, 8)
            pltpu.make_async_copy(idx_hbm.at[pl.ds(off, _CH)], idxs[b], sem_i).wait()
            pltpu.make_async_copy(vals_hbm.at[pl.ds(off, _CH)], bufs[b], sem_l).wait()

        def s_start(b):
            pltpu.async_copy(bufs[b], acc_sh.at[idxs[b]], sem_s, add=True)

        def s_wait(b):
            pltpu.make_async_copy(bufs[b], acc_sh.at[idxs[b]], sem_s).wait()

        for b in range(nb):
            l_start(b, b)

        def body(t, carry):
            j0 = t * nb
            for b in range(nb):
                l_wait(j0 + b, b)
                s_start(b)
            for b in range(nb):
                s_wait(b)
                l_start(j0 + nb + b, b)
            return carry

        lax.fori_loop(0, _NCHUNK // nb - 1, body, 0)
        jl = _NCHUNK - nb
        for b in range(nb):
            l_wait(jl + b, b)
            s_start(b)
        for b in range(nb):
            s_wait(b)
        plsc.subcore_barrier()

        def obody(kk, carry):
            src_off = pl.multiple_of(r0 + kk * _CH, 8)
            dst_off = pl.multiple_of(cid * _NP + r0 + kk * _CH, 8)
            pltpu.sync_copy(acc_sh.at[pl.ds(src_off, _CH)], v0)
            pltpu.sync_copy(v0, out_hbm.at[pl.ds(dst_off, _CH)])
            return carry

        lax.fori_loop(0, _SLAB, obody, 0)

    return k(vals, idx, zeros_h)


def _sc_degree(dst64, zeros_h, ones_h):
    ch = 64
    nchunk = _PW // ch

    @functools.partial(
        pl.kernel,
        mesh=_mesh(),
        out_type=jax.ShapeDtypeStruct((2 * _NP, _H), _f32),
        scratch_types=[
            pltpu.VMEM((nchunk, ch), _i32),
            pltpu.VMEM((ch, _H), _f32),
            pltpu.VMEM_SHARED((_NP, _H), _f32),
        ],
    )
    def k(dst_hbm, zeros_hbm, ones_hbm, out_hbm, didx, buf, acc_sh):
        cid = lax.axis_index("c")
        sid = lax.axis_index("s")
        wid = sid * _NC + cid
        r0 = sid * _RPT
        irow0 = wid * nchunk
        pltpu.sync_copy(dst_hbm.at[pl.ds(irow0, nchunk)], didx)
        pltpu.sync_copy(zeros_hbm.at[pl.ds(0, ch)], buf)

        def zbody(kk, carry):
            pltpu.sync_copy(buf, acc_sh.at[pl.ds(pl.multiple_of(r0 + kk * ch, 8), ch)])
            return carry

        lax.fori_loop(0, _RPT // ch, zbody, 0)
        plsc.subcore_barrier()
        pltpu.sync_copy(ones_hbm, buf)

        def sbody(j, carry):
            pltpu.sync_copy(buf, acc_sh.at[didx.at[j]], add=True)
            return carry

        lax.fori_loop(0, nchunk, sbody, 0)
        plsc.subcore_barrier()

        def obody(kk, carry):
            src_off = pl.multiple_of(r0 + kk * ch, 8)
            dst_off = pl.multiple_of(cid * _NP + r0 + kk * ch, 8)
            pltpu.sync_copy(acc_sh.at[pl.ds(src_off, ch)], buf)
            pltpu.sync_copy(buf, out_hbm.at[pl.ds(dst_off, ch)])
            return carry

        lax.fori_loop(0, _RPT // ch, obody, 0)

    return k(dst64, zeros_h, ones_h)


def _sc_gather_scatter(z, u, src64, dst64, zeros_h):
    D = z.shape[1]
    ch = 64
    nchunk = _PW // ch

    @functools.partial(
        pl.kernel,
        mesh=_mesh(),
        out_type=jax.ShapeDtypeStruct((2 * _NP, D), _f32),
        scratch_types=[
            pltpu.VMEM((nchunk // 2, ch), _i32),
            pltpu.VMEM((nchunk // 2, ch), _i32),
            pltpu.VMEM((ch, D), _f32),
            pltpu.VMEM((ch, D), _f32),
            pltpu.VMEM((ch, D), _f32),
            pltpu.VMEM((ch, D), _f32),
            pltpu.VMEM_SHARED((_NP, D), _f32),
            pltpu.SemaphoreType.DMA,
            pltpu.SemaphoreType.DMA,
            pltpu.SemaphoreType.DMA,
        ],
    )
    def k(z_hbm, u_hbm, src_hbm, dst_hbm, zeros_hbm, out_hbm,
          sidx, didx, g0, g1, u0, u1, acc_sh, sem_g, sem_l, sem_s):
        cid = lax.axis_index("c")
        sid = lax.axis_index("s")
        wid = sid * _NC + cid
        r0 = sid * _RPT
        gbufs = [g0, g1]
        ubufs = [u0, u1]
        nb = 2
        irow0 = wid * nchunk
        pltpu.sync_copy(zeros_hbm.at[pl.ds(0, ch)], g0)

        def zbody(kk, carry):
            pltpu.sync_copy(g0, acc_sh.at[pl.ds(pl.multiple_of(r0 + kk * ch, 8), ch)])
            return carry

        lax.fori_loop(0, _RPT // ch, zbody, 0)
        plsc.subcore_barrier()

        base = pl.multiple_of(wid * _PW, 8)
        nh = nchunk // 2

        for ph in range(2):
            jb = ph * nh
            pltpu.sync_copy(src_hbm.at[pl.ds(irow0 + jb, nh)], sidx)
            pltpu.sync_copy(dst_hbm.at[pl.ds(irow0 + jb, nh)], didx)

            def l_start(j, b, jb=jb):
                off = pl.multiple_of(base + (jb + j) * ch, 8)
                pltpu.async_copy(z_hbm.at[sidx.at[j]], gbufs[b], sem_g)
                pltpu.async_copy(u_hbm.at[pl.ds(off, ch)], ubufs[b], sem_l)

            def l_wait(j, b, jb=jb):
                off = pl.multiple_of(base + (jb + j) * ch, 8)
                pltpu.make_async_copy(z_hbm.at[sidx.at[j]], gbufs[b], sem_g).wait()
                pltpu.make_async_copy(u_hbm.at[pl.ds(off, ch)], ubufs[b], sem_l).wait()

            def s_start(j, b):
                pltpu.async_copy(gbufs[b], acc_sh.at[didx.at[j]], sem_s, add=True)
                pltpu.async_copy(ubufs[b], acc_sh.at[didx.at[j]], sem_s, add=True)

            def s_wait(j, b):
                pltpu.make_async_copy(gbufs[b], acc_sh.at[didx.at[j]], sem_s).wait()
                pltpu.make_async_copy(ubufs[b], acc_sh.at[didx.at[j]], sem_s).wait()

            for b in range(nb):
                l_start(b, b)

            def body(t, carry, l_start=l_start, l_wait=l_wait,
                     s_start=s_start, s_wait=s_wait):
                j0 = t * nb
                for b in range(nb):
                    l_wait(j0 + b, b)
                    s_start(j0 + b, b)
                for b in range(nb):
                    s_wait(j0 + b, b)
                    l_start(j0 + nb + b, b)
                return carry

            lax.fori_loop(0, nh // nb - 1, body, 0)
            jl = nh - nb
            for b in range(nb):
                l_wait(jl + b, b)
                s_start(jl + b, b)
            for b in range(nb):
                s_wait(jl + b, b)
        plsc.subcore_barrier()

        def obody(kk, carry):
            src_off = pl.multiple_of(r0 + kk * ch, 8)
            dst_off = pl.multiple_of(cid * _NP + r0 + kk * ch, 8)
            pltpu.sync_copy(acc_sh.at[pl.ds(src_off, ch)], g0)
            pltpu.sync_copy(g0, out_hbm.at[pl.ds(dst_off, ch)])
            return carry

        lax.fori_loop(0, _RPT // ch, obody, 0)

    return k(z, u, src64, dst64, zeros_h)


def _cparams():
    return pltpu.CompilerParams(dimension_semantics=("arbitrary",))


def _onehot_dot(idx, table_ref, ncls):
    oh = (idx[:, None] == lax.broadcasted_iota(_i32, (idx.shape[0], ncls), 1)).astype(_f32)
    return jax.lax.dot_general(oh, table_ref[...], (((1,), (0,)), ((), ())),
                               precision=jax.lax.Precision.HIGHEST,
                               preferred_element_type=_f32)


def _embed_nodes(x0, x1, nmw, nmm, a0p, a1p, wm0):
    def body(x0_r, x1_r, mw_r, mm_r, a0_r, a1_r, wm_r, ow_r, om_r, ozw_r, ozm_r):
        i0 = x0_r[0, 0, :]
        i1 = x1_r[0, 0, :]
        h = _onehot_dot(i0, a0_r, 128) + _onehot_dot(i1, a1_r, 128)
        hw = h * (1.0 - mw_r[0, 0, :])[:, None]
        hm = h * (1.0 - mm_r[0, 0, :])[:, None]
        ow_r[...] = hw
        om_r[...] = hm
        ozw_r[...] = jnp.dot(hw, wm_r[...], preferred_element_type=_f32)
        ozm_r[...] = jnp.dot(hm, wm_r[...], preferred_element_type=_f32)

    sh = jax.ShapeDtypeStruct((_NP, _H), _f32)
    v3 = pl.BlockSpec((1, 1, _RN), lambda i: (i, 0, 0))
    full = pl.BlockSpec((128, _H), lambda i: (0, 0))
    blkh = pl.BlockSpec((_RN, _H), lambda i: (i, 0))
    return pl.pallas_call(
        body, grid=(_GN,),
        in_specs=[v3, v3, v3, v3, full, full, full],
        out_specs=[blkh] * 4,
        out_shape=[sh] * 4, compiler_params=_cparams(),
    )(x0, x1, nmw, nmm, a0p, a1p, wm0)


def _embed_edges(e0, e1, emw, emm, b0p, b1p, wm0):
    def body(e0_r, e1_r, mw_r, mm_r, b0_r, b1_r, wm_r, ow_r, om_r, ouw_r, oum_r):
        i0 = e0_r[0, 0, :]
        i1 = e1_r[0, 0, :]
        h = _onehot_dot(i0, b0_r, 32) + _onehot_dot(i1, b1_r, 32)
        ew = h * (1.0 - mw_r[0, 0, :])[:, None]
        em = h * (1.0 - mm_r[0, 0, :])[:, None]
        ow_r[...] = ew
        om_r[...] = em
        ouw_r[...] = jnp.dot(ew, wm_r[...], preferred_element_type=_f32)
        oum_r[...] = jnp.dot(em, wm_r[...], preferred_element_type=_f32)

    sh = jax.ShapeDtypeStruct((_EP, _H), _f32)
    v3 = pl.BlockSpec((1, 1, _RE), lambda i: (i, 0, 0))
    full = pl.BlockSpec((32, _H), lambda i: (0, 0))
    wfull = pl.BlockSpec((_H, _H), lambda i: (0, 0))
    blk = pl.BlockSpec((_RE, _H), lambda i: (i, 0))
    return pl.pallas_call(
        body, grid=(_GE,),
        in_specs=[v3, v3, v3, v3, full, full, wfull],
        out_specs=[blk] * 4,
        out_shape=[sh] * 4, compiler_params=_cparams(),
    )(e0, e1, emw, emm, b0p, b1p, wm0)


def _node_update(h, parts, degp, w, wm_next, emit_z):
    def body(h_r, p0_r, p1_r, d0_r, d1_r, w_r, wm_r, o_r, *outs):
        agg = p0_r[...] + p1_r[...]
        deg = d0_r[:, 0:1] + d1_r[:, 0:1]
        inv = 1.0 / jnp.maximum(deg, 1.0)
        hn = jnp.maximum(
            jnp.dot(h_r[...], w_r[...], preferred_element_type=_f32) + agg * inv, 0.0)
        o_r[...] = hn
        if emit_z:
            outs[0][...] = jnp.dot(hn, wm_r[...], preferred_element_type=_f32)

    blk = pl.BlockSpec((_RN, _H), lambda i: (i, 0))
    p0 = pl.BlockSpec((_RN, _H), lambda i: (i, 0))
    p1 = pl.BlockSpec((_RN, _H), lambda i: (i + _GN, 0))
    d0 = pl.BlockSpec((_RN, _H), lambda i: (i, 0))
    d1 = pl.BlockSpec((_RN, _H), lambda i: (i + _GN, 0))
    full = pl.BlockSpec((_H, _H), lambda i: (0, 0))
    sh = jax.ShapeDtypeStruct((_NP, _H), _f32)
    out_specs = [blk] + ([blk] if emit_z else [])
    out_shape = [sh] + ([sh] if emit_z else [])
    return pl.pallas_call(
        body, grid=(_GN,), in_specs=[blk, p0, p1, d0, d1, full, full],
        out_specs=out_specs, out_shape=out_shape,
        compiler_params=_cparams(),
    )(h, parts, parts, degp, degp, w, wm_next)


def _edge_update(ea, gc, w, wm_next, emit_u):
    def body(ea_r, ga_r, gb_r, w_r, wm_r, o_r, *outs):
        en = jnp.maximum(
            jnp.dot(ea_r[...], w_r[...], preferred_element_type=_f32)
            + ga_r[...] + gb_r[...], 0.0)
        o_r[...] = en
        if emit_u:
            outs[0][...] = jnp.dot(en, wm_r[...], preferred_element_type=_f32)

    blk = pl.BlockSpec((_RE, _H), lambda i: (i, 0))
    blka = pl.BlockSpec((_RE, _H), lambda i: (i, 0))
    blkb = pl.BlockSpec((_RE, _H), lambda i: (i + _GE, 0))
    full = pl.BlockSpec((_H, _H), lambda i: (0, 0))
    sh = jax.ShapeDtypeStruct((_EP, _H), _f32)
    out_specs = [blk] + ([blk] if emit_u else [])
    out_shape = [sh] + ([sh] if emit_u else [])
    return pl.pallas_call(
        body, grid=(_GE,), in_specs=[blk, blka, blkb, full, full],
        out_specs=out_specs, out_shape=out_shape, compiler_params=_cparams(),
    )(ea, gc, gc, w, wm_next)


def _acc2(ref, val, i):
    @pl.when(i == 0)
    def _():
        ref[...] = jnp.zeros((1, 1), _f32)

    ref[...] += jnp.reshape(val, (1, 1))


def _ce_from_logits(z, lab, ncls_real):
    cm = lax.broadcasted_iota(_i32, z.shape, 1) < ncls_real
    z = jnp.where(cm, z, -1e30)
    m = jnp.max(z, axis=1, keepdims=True)
    lse = m + jnp.log(jnp.sum(jnp.exp(z - m), axis=1, keepdims=True))
    oh = lab[:, None] == lax.broadcasted_iota(_i32, z.shape, 1)
    zy = jnp.sum(jnp.where(oh, z, 0.0), axis=1, keepdims=True)
    return lse - zy


def _node_ce(x1, x2, x3, wn3, bn, lab3, w3):
    def body(x1_r, x2_r, x3_r, w_r, b_r, lab_r, wt_r, s_r, sw_r):
        i = pl.program_id(0)
        z = (jnp.dot(x1_r[...], w_r[0], preferred_element_type=_f32)
             + jnp.dot(x2_r[...], w_r[1], preferred_element_type=_f32)
             + jnp.dot(x3_r[...], w_r[2], preferred_element_type=_f32)
             + b_r[...])
        nll = _ce_from_logits(z, lab_r[0, 0, :], _NA)
        wt = wt_r[0, 0, :][:, None]
        _acc2(s_r, jnp.sum(nll * wt), i)
        _acc2(sw_r, jnp.sum(wt), i)

    blk = pl.BlockSpec((_RN, _H), lambda i: (i, 0))
    wfull = pl.BlockSpec((3, _H, 128), lambda i: (0, 0, 0))
    bfull = pl.BlockSpec((1, 128), lambda i: (0, 0))
    v3 = pl.BlockSpec((1, 1, _RN), lambda i: (i, 0, 0))
    sc = pl.BlockSpec((1, 1), lambda i: (0, 0))
    s1 = jax.ShapeDtypeStruct((1, 1), _f32)
    return pl.pallas_call(
        body, grid=(_GN,), in_specs=[blk, blk, blk, wfull, bfull, v3, v3],
        out_specs=[sc, sc], out_shape=[s1, s1], compiler_params=_cparams(),
    )(x1, x2, x3, wn3, bn, lab3, w3)


def _edge_ce(es, gcs, we3, be, lab3, w3, emit_e):
    def body(e1_r, e2_r, e3_r, a1_r, a2_r, a3_r, b1_r, b2_r, b3_r,
             w_r, b_r, lab_r, wt_r, *outs):
        i = pl.program_id(0)
        c1 = e1_r[...] + a1_r[...] + b1_r[...]
        c2 = e2_r[...] + a2_r[...] + b2_r[...]
        c3 = e3_r[...] + a3_r[...] + b3_r[...]
        z = (jnp.dot(c1, w_r[0], preferred_element_type=_f32)
             + jnp.dot(c2, w_r[1], preferred_element_type=_f32)
             + jnp.dot(c3, w_r[2], preferred_element_type=_f32)
             + b_r[...])
        nll = _ce_from_logits(z, lab_r[0, 0, :], _NB)
        wt = wt_r[0, 0, :][:, None]
        _acc2(outs[0], jnp.sum(nll * wt), i)
        _acc2(outs[1], jnp.sum(wt), i)
        if emit_e:
            outs[2][...] = c1
            outs[3][...] = c2
            outs[4][...] = c3

    blk = pl.BlockSpec((_RE, _H), lambda i: (i, 0))
    blkb = pl.BlockSpec((_RE, _H), lambda i: (i + _GE, 0))
    wfull = pl.BlockSpec((3, _H, 32), lambda i: (0, 0, 0))
    bfull = pl.BlockSpec((1, 32), lambda i: (0, 0))
    v3 = pl.BlockSpec((1, 1, _RE), lambda i: (i, 0, 0))
    sc = pl.BlockSpec((1, 1), lambda i: (0, 0))
    s1 = jax.ShapeDtypeStruct((1, 1), _f32)
    eo = jax.ShapeDtypeStruct((_EP, _H), _f32)
    out_specs = [sc, sc] + ([blk] * 3 if emit_e else [])
    out_shape = [s1, s1] + ([eo] * 3 if emit_e else [])
    return pl.pallas_call(
        body, grid=(_GE,),
        in_specs=[blk] * 3 + [blk] * 3 + [blkb] * 3 + [wfull, bfull, v3, v3],
        out_specs=out_specs, out_shape=out_shape, compiler_params=_cparams(),
    )(es[0], es[1], es[2], gcs[0], gcs[1], gcs[2], gcs[0], gcs[1], gcs[2],
      we3, be, lab3, w3)


def _bce_elem(z, y):
    return jnp.maximum(z, 0.0) - z * y + jnp.log(1.0 + jnp.exp(-jnp.abs(z)))


def _node_comp(x1, x2, x3, wf0, bf0, wf1, bf1, ws0, bs0, ws1, bs1,
               labfg, labsc3, nmm13):
    def body(x1_r, x2_r, x3_r, wf0_r, bf0_r, wf1_r, bf1_r,
             ws0_r, bs0_r, ws1_r, bs1_r, yfg_r, ysc_r, m_r,
             sfg_r, ssc_r, sw_r):
        i = pl.program_id(0)
        w = (1.0 - m_r[0, 0, :])[:, None]

        hf = (jnp.dot(x1_r[...], wf0_r[0], preferred_element_type=_f32)
              + jnp.dot(x2_r[...], wf0_r[1], preferred_element_type=_f32)
              + jnp.dot(x3_r[...], wf0_r[2], preferred_element_type=_f32)
              + bf0_r[...])
        hf = jnp.maximum(hf, 0.0)
        zf = jnp.dot(hf, wf1_r[...], preferred_element_type=_f32) + bf1_r[...]
        cmf = (lax.broadcasted_iota(_i32, zf.shape, 1) < _FG).astype(_f32)
        perf = jnp.sum(_bce_elem(zf, yfg_r[...]) * cmf, axis=1, keepdims=True) / _FG

        hs = (jnp.dot(x1_r[...], ws0_r[0], preferred_element_type=_f32)
              + jnp.dot(x2_r[...], ws0_r[1], preferred_element_type=_f32)
              + jnp.dot(x3_r[...], ws0_r[2], preferred_element_type=_f32)
              + bs0_r[...])
        hs = jnp.maximum(hs, 0.0)
        zs = jnp.dot(hs, ws1_r[...], preferred_element_type=_f32) + bs1_r[...]
        ysc = ysc_r[0, 0, :][:, None]
        pers = _bce_elem(zs[:, 0:1], ysc)
        _acc2(sfg_r, jnp.sum(perf * w), i)
        _acc2(ssc_r, jnp.sum(pers * w), i)
        _acc2(sw_r, jnp.sum(w), i)

    blk = pl.BlockSpec((_RN, _H), lambda i: (i, 0))
    w0full = pl.BlockSpec((3, _H, 384), lambda i: (0, 0, 0))
    b0full = pl.BlockSpec((1, 384), lambda i: (0, 0))
    w1full = pl.BlockSpec((384, 128), lambda i: (0, 0))
    b1full = pl.BlockSpec((1, 128), lambda i: (0, 0))
    yfull = pl.BlockSpec((_RN, 128), lambda i: (i, 0))
    v3 = pl.BlockSpec((1, 1, _RN), lambda i: (i, 0, 0))
    sc = pl.BlockSpec((1, 1), lambda i: (0, 0))
    s1 = jax.ShapeDtypeStruct((1, 1), _f32)
    return pl.pallas_call(
        body, grid=(_GN,),
        in_specs=[blk, blk, blk, w0full, b0full, w1full, b1full,
                  w0full, b0full, w1full, b1full, yfull, v3, v3],
        out_specs=[sc, sc, sc], out_shape=[s1, s1, s1], compiler_params=_cparams(),
    )(x1, x2, x3, wf0, bf0, wf1, bf1, ws0, bs0, ws1, bs1, labfg, labsc3, nmm13)


def _edge_comp(e1, e2, e3, wb0, bb0, wb1, bb1, labbr3, emm13):
    def body(e1_r, e2_r, e3_r, w0_r, b0_r, w1_r, b1_r, y_r, m_r, s_r, sw_r):
        i = pl.program_id(0)
        w = (1.0 - m_r[0, 0, :])[:, None]
        hh = (jnp.dot(e1_r[...], w0_r[0], preferred_element_type=_f32)
              + jnp.dot(e2_r[...], w0_r[1], preferred_element_type=_f32)
              + jnp.dot(e3_r[...], w0_r[2], preferred_element_type=_f32)
              + b0_r[...])
        hh = jnp.maximum(hh, 0.0)
        z = jnp.dot(hh, w1_r[...], preferred_element_type=_f32) + b1_r[...]
        y = y_r[0, 0, :][:, None]
        per = _bce_elem(z[:, 0:1], y)
        _acc2(s_r, jnp.sum(per * w), i)
        _acc2(sw_r, jnp.sum(w), i)

    blk = pl.BlockSpec((_RE, _H), lambda i: (i, 0))
    w0full = pl.BlockSpec((3, _H, 384), lambda i: (0, 0, 0))
    b0full = pl.BlockSpec((1, 384), lambda i: (0, 0))
    w1full = pl.BlockSpec((384, 128), lambda i: (0, 0))
    b1full = pl.BlockSpec((1, 128), lambda i: (0, 0))
    v3 = pl.BlockSpec((1, 1, _RE), lambda i: (i, 0, 0))
    sc = pl.BlockSpec((1, 1), lambda i: (0, 0))
    s1 = jax.ShapeDtypeStruct((1, 1), _f32)
    return pl.pallas_call(
        body, grid=(_GE,),
        in_specs=[blk, blk, blk, w0full, b0full, w1full, b1full, v3, v3],
        out_specs=[sc, sc], out_shape=[s1, s1], compiler_params=_cparams(),
    )(e1, e2, e3, wb0, bb0, wb1, bb1, labbr3, emm13)


def _padn(v, fill=0):
    return jnp.pad(v, ((0, _NP - _N),) + ((0, 0),) * (v.ndim - 1), constant_values=fill)


def _pade(v, fill=0):
    return jnp.pad(v, ((0, _EP - _E),) + ((0, 0),) * (v.ndim - 1), constant_values=fill)


def _r3(v, r):
    return v.reshape(-1, 1, r)


def kernel(x, edge_attr, edge_index, node_mask, edge_mask, node_mask_motif,
           edge_mask_motif, label_fg, label_brics, label_scaffold, params):
    p = params

    x0 = _r3(_padn(x[:, 0].astype(_i32)), _RN)
    x1 = _r3(_padn(x[:, 1].astype(_i32)), _RN)
    e0 = _r3(_pade(edge_attr[:, 0].astype(_i32)), _RE)
    e1 = _r3(_pade(edge_attr[:, 1].astype(_i32)), _RE)
    src = _pade(edge_index[0].astype(_i32), 0)
    dst = _pade(edge_index[1].astype(_i32), _NP - 1)
    srcdst2 = jnp.concatenate([src.reshape(-1, _CH), dst.reshape(-1, _CH)])
    src64 = src.reshape(-1, 64)
    dst64 = dst.reshape(-1, 64)

    nmw = _r3(_padn(node_mask.astype(_f32)), _RN)
    nmm0 = _r3(_padn(node_mask_motif.astype(_f32)), _RN)
    nmm1 = _r3(_padn(node_mask_motif.astype(_f32), 1.0), _RN)
    emw = _r3(_pade(edge_mask.astype(_f32)), _RE)
    emm0 = _r3(_pade(edge_mask_motif.astype(_f32)), _RE)
    emm1 = _r3(_pade(edge_mask_motif.astype(_f32), 1.0), _RE)

    labn = x0
    labe = e0
    labfg = jnp.pad(_padn(label_fg), ((0, 0), (0, 128 - _FG)))
    labsc = _r3(_padn(label_scaffold[:, 0]), _RN)
    labbr = _r3(_pade(label_brics[:, 0]), _RE)

    a0p = jnp.pad(p['emb_a0'], ((0, 128 - _NA), (0, 0)))
    a1p = jnp.pad(p['emb_a1'], ((0, 128 - _NA), (0, 0)))
    b0p = jnp.pad(p['emb_b0'], ((0, 32 - _NB), (0, 0)))
    b1p = jnp.pad(p['emb_b1'], ((0, 32 - _NB), (0, 0)))

    wn3 = jnp.pad(p['Wn'], ((0, 0), (0, 128 - _NA))).reshape(3, _H, 128)
    bn = jnp.pad(p['bn'], (0, 128 - _NA)).reshape(1, 128)
    we3 = jnp.pad(p['We'], ((0, 0), (0, 32 - _NB))).reshape(3, _H, 32)
    be = jnp.pad(p['be'], (0, 32 - _NB)).reshape(1, 32)

    wf0 = p['W_fg_0'].reshape(3, _H, 384)
    bf0 = p['b_fg_0'].reshape(1, 384)
    wf1 = jnp.pad(p['W_fg_1'], ((0, 0), (0, 128 - _FG)))
    bf1 = jnp.pad(p['b_fg_1'], (0, 128 - _FG)).reshape(1, 128)
    ws0 = p['W_scaffold_0'].reshape(3, _H, 384)
    bs0 = p['b_scaffold_0'].reshape(1, 384)
    ws1 = jnp.pad(p['W_scaffold_1'], ((0, 0), (0, 127)))
    bs1 = jnp.pad(p['b_scaffold_1'], (0, 127)).reshape(1, 128)
    wb0 = p['W_brics_0'].reshape(3, _H, 384)
    bb0 = p['b_brics_0'].reshape(1, 384)
    wb1 = jnp.pad(p['W_brics_1'], ((0, 0), (0, 127)))
    bb1 = jnp.pad(p['b_brics_1'], (0, 127)).reshape(1, 128)

    zeros_h = jnp.zeros((_CH, _H), _f32)

    wm = [p['W_msg%d' % i] for i in range(_L)]
    hw, hm, zw, zm = _embed_nodes(x0, x1, nmw, nmm0, a0p, a1p, wm[0])
    eaw, eam, uw, um = _embed_edges(e0, e1, emw, emm0, b0p, b1p, wm[0])
    degp = _sc_degree(dst64, zeros_h, jnp.ones((64, _H), _f32))

    def run_pass(h, z, ea, u, emit_e, nmask, emask):
        xs, gcs, es = [], [], []
        for i in range(_L):
            emit = i < _L - 1
            wm_next = wm[i + 1] if emit else wm[i]
            parts = _sc_gather_scatter(z, u, src64, dst64, zeros_h)
            nouts = _node_update(h, parts, degp, p['W_self%d' % i], wm_next, emit)
            h = nouts[0]
            gc = _sc_gather(h, srcdst2)
            uouts = _edge_update(ea, gc, p['W_edge%d' % i], wm_next, emit)
            ea = uouts[0]
            if emit:
                z, u = nouts[1], uouts[1]
            xs.append(h); es.append(ea); gcs.append(gc)
        sn, swn = _node_ce(xs[0], xs[1], xs[2], wn3, bn, labn, nmask)
        eouts = _edge_ce(es, gcs, we3, be, labe, emask, emit_e)
        return xs, eouts, sn, swn

    xs_w, eo_w, sn_w, swn_w = run_pass(hw, zw, eaw, uw, False, nmw, emw)
    se_w, swe_w = eo_w[0], eo_w[1]
    xs_m, eo_m, sn_m, swn_m = run_pass(hm, zm, eam, um, True, nmm0, emm0)
    se_m, swe_m = eo_m[0], eo_m[1]
    ec1, ec2, ec3 = eo_m[2], eo_m[3], eo_m[4]

    sfg, ssc, swc = _node_comp(xs_m[0], xs_m[1], xs_m[2], wf0, bf0, wf1, bf1,
                               ws0, bs0, ws1, bs1, labfg, labsc, nmm1)
    sbr, swbr = _edge_comp(ec1, ec2, ec3, wb0, bb0, wb1, bb1, labbr, emm1)

    def _div(a, b):
        return (a[0, 0] / jnp.maximum(b[0, 0], 1.0)).astype(_f32)

    ln_w = _div(sn_w, swn_w)
    le_w = _div(se_w, swe_w)
    ln_m = _div(sn_m, swn_m)
    le_m = _div(se_m, swe_m)
    l_fg = _div(sfg, swc)
    l_sc = _div(ssc, swc)
    l_br = _div(sbr, swbr)

    sep = jnp.stack([ln_w, le_w, ln_m, le_m, l_fg, l_br, l_sc])
    loss = ln_w + le_w + ln_m + le_m + l_fg + l_br + l_sc
    return (loss, sep)
